# Initial kernel scaffold; baseline (speedup 1.0000x reference)
import jax
import jax.numpy as jnp
from jax.experimental import pallas as pl


def kernel(xyz, points, W1, b1, W2, b2, W3, b3):
    raise NotImplementedError("write your pallas kernel here")


# speedup vs baseline: 11.1757x; 11.1757x over previous
import functools

import jax
import jax.numpy as jnp
import numpy as np
from jax import lax
from jax.experimental import pallas as pl
from jax.experimental.pallas import tpu as pltpu
from jax.experimental.pallas import tpu_sc as plsc

NPOINT = 512
RADIUS = 0.2
NSAMPLE = 64
R2 = float(RADIUS) ** 2

SC_CORES = 2
SC_SUBCORES = 16
SC_WORKERS = SC_CORES * SC_SUBCORES
SC_LANES = 16


def _fps_body(xyz_ref, cent_ref, nx_ref):
    x = xyz_ref[...]
    bsz, _, n = x.shape
    x0 = x[:, 0, :]
    x1 = x[:, 1, :]
    x2 = x[:, 2, :]
    m0 = jnp.mean(x0, axis=1, keepdims=True)
    m1 = jnp.mean(x1, axis=1, keepdims=True)
    m2 = jnp.mean(x2, axis=1, keepdims=True)
    dist0 = (x0 - m0) ** 2 + (x1 - m1) ** 2 + (x2 - m2) ** 2
    distance = jnp.minimum(jnp.full((bsz, n), 1e10, jnp.float32), dist0)
    lane = lax.broadcasted_iota(jnp.int32, (bsz, n), 1)
    scol = lax.broadcasted_iota(jnp.int32, (bsz, NPOINT), 1)

    def argmax_rows(d):
        mv = jnp.max(d, axis=1, keepdims=True)
        return jnp.min(jnp.where(d == mv, lane, n), axis=1, keepdims=True)

    far = argmax_rows(distance)
    cent = jnp.zeros((bsz, NPOINT), jnp.int32)
    nx0 = jnp.zeros((bsz, NPOINT), jnp.float32)
    nx1 = jnp.zeros((bsz, NPOINT), jnp.float32)
    nx2 = jnp.zeros((bsz, NPOINT), jnp.float32)

    def body(i, carry):
        distance, far, cent, nx0, nx1, nx2 = carry
        oh = lane == far
        c0 = jnp.sum(jnp.where(oh, x0, 0.0), axis=1, keepdims=True)
        c1 = jnp.sum(jnp.where(oh, x1, 0.0), axis=1, keepdims=True)
        c2 = jnp.sum(jnp.where(oh, x2, 0.0), axis=1, keepdims=True)
        sel = scol == i
        cent = jnp.where(sel, far, cent)
        nx0 = jnp.where(sel, c0, nx0)
        nx1 = jnp.where(sel, c1, nx1)
        nx2 = jnp.where(sel, c2, nx2)
        d = (x0 - c0) ** 2 + (x1 - c1) ** 2 + (x2 - c2) ** 2
        distance = jnp.minimum(distance, d)
        far = argmax_rows(distance)
        return (distance, far, cent, nx0, nx1, nx2)

    carry = (distance, far, cent, nx0, nx1, nx2)
    _, _, cent, nx0, nx1, nx2 = lax.fori_loop(0, NPOINT, body, carry)
    cent_ref[...] = cent
    nx_ref[:, 0, :] = nx0
    nx_ref[:, 1, :] = nx1
    nx_ref[:, 2, :] = nx2


def _fps(xyz):
    bsz = xyz.shape[0]
    return pl.pallas_call(
        _fps_body,
        out_shape=[
            jax.ShapeDtypeStruct((bsz, NPOINT), jnp.int32),
            jax.ShapeDtypeStruct((bsz, 3, NPOINT), jnp.float32),
        ],
    )(xyz)


def _g1_body(xyz_ref, pts_ref, nx_ref, w1_ref, b1_ref, g1_ref, u1_ref):
    pts = pts_ref[0]
    xyzrow = xyz_ref[0, 0]
    nxrow = nx_ref[0, 0]
    w1 = w1_ref[...]
    b1 = b1_ref[...]
    w01 = w1[:, 0] + w1[:, 1]
    w1p = w1[:, 2:]
    g = lax.dot_general(pts, w1p, (((0,), (1,)), ((), ())),
                        preferred_element_type=jnp.float32)
    g = g + xyzrow[:, None] * w01[None, :]
    g1_ref[0] = g
    u1_ref[0] = b1 - nxrow[:, None] * w1[:, 0][None, :]


def _g1_u1(xyz, points, nxyz, W1, b1):
    b3, d, n = points.shape
    c1 = W1.shape[0]
    return pl.pallas_call(
        _g1_body,
        grid=(b3,),
        in_specs=[
            pl.BlockSpec((1, 1, n), lambda i: (i, 0, 0)),
            pl.BlockSpec((1, d, n), lambda i: (i, 0, 0)),
            pl.BlockSpec((1, 1, NPOINT), lambda i: (i, 0, 0)),
            pl.BlockSpec((c1, d + 2), lambda i: (0, 0)),
            pl.BlockSpec((1, c1), lambda i: (0, 0)),
        ],
        out_specs=[
            pl.BlockSpec((1, n, c1), lambda i: (i, 0, 0)),
            pl.BlockSpec((1, NPOINT, c1), lambda i: (i, 0, 0)),
        ],
        out_shape=[
            jax.ShapeDtypeStruct((b3, n, c1), jnp.float32),
            jax.ShapeDtypeStruct((b3, NPOINT, c1), jnp.float32),
        ],
    )(xyz.reshape(b3, 1, n), points, nxyz.reshape(b3, 1, NPOINT),
      W1, b1.reshape(1, c1))


def _flags_body(nx_ref, xt_ref, f_ref):
    src = nx_ref[0]
    dst = xt_ref[0]
    mm = lax.dot_general(src, dst, (((1,), (1,)), ((), ())))
    dist = -2.0 * mm
    dist = dist + jnp.sum(src ** 2, -1)[:, None]
    dist = dist + jnp.sum(dst ** 2, -1)[None, :]
    f_ref[0] = (dist <= R2).astype(jnp.int32)


def _flags(nx_t, xyz_t):
    bsz, n, _ = xyz_t.shape
    return pl.pallas_call(
        _flags_body,
        grid=(bsz,),
        in_specs=[pl.BlockSpec((1, NPOINT, 3), lambda i: (i, 0, 0)),
                  pl.BlockSpec((1, n, 3), lambda i: (i, 0, 0))],
        out_specs=pl.BlockSpec((1, NPOINT, n), lambda i: (i, 0, 0)),
        out_shape=jax.ShapeDtypeStruct((bsz, NPOINT, n), jnp.int32),
    )(nx_t, xyz_t)


def _select_sc(flags):
    rows = flags.shape[0] * flags.shape[1]
    n = flags.shape[2]
    flags = flags.reshape(rows, n)
    rows_per_w = rows // SC_WORKERS
    slab_rows = 16
    nslabs = rows_per_w // slab_rows
    nchunks = n // SC_LANES
    mesh = plsc.VectorSubcoreMesh(core_axis_name="c", subcore_axis_name="s",
                                  num_cores=SC_CORES, num_subcores=SC_SUBCORES)

    @functools.partial(
        pl.kernel,
        out_type=jax.ShapeDtypeStruct((rows, NSAMPLE), jnp.int32),
        mesh=mesh,
        compiler_params=pltpu.CompilerParams(needs_layout_passes=False),
        scratch_types=[
            pltpu.VMEM((slab_rows, n), jnp.int32),
            pltpu.VMEM((rows_per_w, NSAMPLE), jnp.int32),
        ],
    )
    def sel(f_hbm, idx_hbm, slabv, bufv):
        wid = lax.axis_index("s") * SC_CORES + lax.axis_index("c")
        lanes = lax.broadcasted_iota(jnp.int32, (SC_LANES,), 0)
        zeros = jnp.zeros((SC_LANES,), jnp.int32)

        def slab_body(sl, _):
            pltpu.sync_copy(
                f_hbm.at[pl.ds(wid * rows_per_w + sl * slab_rows, slab_rows)],
                slabv)

            def row_body(j2, _):
                rowv = jnp.full((SC_LANES,), j2, jnp.int32)
                browv = sl * slab_rows + rowv

                def cond(carry):
                    t, cnt = carry
                    return jnp.logical_and(t < nchunks, cnt < NSAMPLE)

                def step(carry):
                    t, cnt = carry
                    col = t * SC_LANES + lanes
                    mi = plsc.load_gather(slabv, [rowv, col])
                    m = mi == 1
                    incl = plsc.cumsum(mi)
                    pos = cnt + incl - mi
                    mstore = jnp.logical_and(m, pos < NSAMPLE)
                    plsc.store_scatter(bufv, [browv, pos], col, mask=mstore)
                    return (t + 1, cnt + jnp.max(incl))

                _, cnt = lax.while_loop(cond, step,
                                        (jnp.int32(0), jnp.int32(0)))

                firstv = plsc.load_gather(bufv, [browv, zeros])
                for mch in range(NSAMPLE // SC_LANES):
                    slots = mch * SC_LANES + lanes
                    cur = plsc.load_gather(bufv, [browv, slots])
                    plsc.store_scatter(bufv, [browv, slots],
                                       jnp.where(slots < cnt, cur, firstv))
                return 0

            lax.fori_loop(0, slab_rows, row_body, 0)
            return 0

        lax.fori_loop(0, nslabs, slab_body, 0)
        pltpu.sync_copy(bufv, idx_hbm.at[pl.ds(wid * rows_per_w, rows_per_w)])

    return sel(flags)


def _gather_sc(g1flat, idxflat):
    total, c1 = g1flat.shape[0], g1flat.shape[1]
    nrows = idxflat.shape[0]
    rows_per_w = nrows // SC_WORKERS
    chunk = 128
    inner = 8
    slab = chunk * inner
    outer = rows_per_w // slab
    mesh = plsc.VectorSubcoreMesh(core_axis_name="c", subcore_axis_name="s",
                                  num_cores=SC_CORES, num_subcores=SC_SUBCORES)

    @functools.partial(
        pl.kernel,
        out_type=jax.ShapeDtypeStruct((nrows, c1), jnp.float32),
        mesh=mesh,
        compiler_params=pltpu.CompilerParams(needs_layout_passes=False,
                                             use_tc_tiling_on_sc=False),
        scratch_types=[
            pltpu.VMEM((rows_per_w,), jnp.int32),
            pltpu.VMEM((slab, c1), jnp.float32),
            pltpu.SemaphoreType.DMA,
        ],
    )
    def gat(tab_hbm, idx_hbm, out_hbm, idxv, rowsv, sem):
        wid = lax.axis_index("s") * SC_CORES + lax.axis_index("c")
        base = wid * rows_per_w
        pltpu.sync_copy(idx_hbm.at[pl.ds(base, rows_per_w)], idxv)

        def outer_body(o, _):
            handles = []
            for k in range(inner):
                src = tab_hbm.at[idxv.at[pl.ds(o * slab + k * chunk, chunk)]]
                dst = rowsv.at[pl.ds(k * chunk, chunk)]
                handles.append(pltpu.async_copy(src, dst, sem))
            for h in handles:
                h.wait()
            pltpu.sync_copy(rowsv, out_hbm.at[pl.ds(base + o * slab, slab)])
            return 0

        lax.fori_loop(0, outer, outer_body, 0)

    return gat(g1flat, idxflat)


def _qrelu_triple(y):
    q = jnp.sqrt(y[0] * y[0] + y[1] * y[1] + y[2] * y[2])
    coef = q / jnp.maximum(q, 1.0)
    return y * coef[None]


def _stats1_body(x_ref, u_ref, ss_ref):
    i = pl.program_id(0)
    j = pl.program_id(1)

    @pl.when(jnp.logical_and(i == 0, j == 0))
    def _():
        ss_ref[...] = jnp.zeros_like(ss_ref)

    x = x_ref[0]
    u = u_ref[0]
    sch, c = u.shape
    y = x.reshape(sch, NSAMPLE, c) + u[:, None, :]
    ss_ref[...] += jnp.sum(y * y, axis=(0, 1)).reshape(1, c)


def _stats1(x1, u1, sch):
    b3, nrows, c1 = x1.shape
    jgrid = NPOINT // sch
    return pl.pallas_call(
        _stats1_body,
        grid=(b3, jgrid),
        in_specs=[
            pl.BlockSpec((1, sch * NSAMPLE, c1), lambda i, j: (i, j, 0)),
            pl.BlockSpec((1, sch, c1), lambda i, j: (i, j, 0)),
        ],
        out_specs=pl.BlockSpec((1, c1), lambda i, j: (0, 0)),
        out_shape=jax.ShapeDtypeStruct((1, c1), jnp.float32),
    )(x1, u1)


def _layer2_body(x_ref, u_ref, ss1_ref, w2_ref, b2_ref, y2_ref, ss2_ref, *,
                 cnt1):
    b = pl.program_id(0)
    j = pl.program_id(1)

    @pl.when(jnp.logical_and(b == 0, j == 0))
    def _():
        ss2_ref[...] = jnp.zeros_like(ss2_ref)

    s1 = lax.rsqrt(ss1_ref[0] / cnt1 + 1e-5)
    x = x_ref[...]
    u = u_ref[...]
    _, sch, c1 = u.shape
    y1 = x.reshape(3, sch, NSAMPLE, c1) + u[:, :, None, :]
    y1 = y1 * s1[None, None, None, :]
    z1 = _qrelu_triple(y1).reshape(3, sch * NSAMPLE, c1)
    w2 = w2_ref[...]
    b2 = b2_ref[...]
    for c in range(3):
        y2 = lax.dot_general(z1[c], w2, (((1,), (1,)), ((), ())),
                             preferred_element_type=jnp.float32) + b2
        y2_ref[c] = y2
        ss2_ref[...] += jnp.sum(y2 * y2, axis=0).reshape(1, -1)


def _layer2(x1, u1, ss1, W2, b2, sch, cnt1):
    b3, nrows, c1 = x1.shape
    c2 = W2.shape[0]
    bsz = b3 // 3
    jgrid = NPOINT // sch
    return pl.pallas_call(
        functools.partial(_layer2_body, cnt1=cnt1),
        grid=(bsz, jgrid),
        in_specs=[
            pl.BlockSpec((3, sch * NSAMPLE, c1), lambda b, j: (b, j, 0)),
            pl.BlockSpec((3, sch, c1), lambda b, j: (b, j, 0)),
            pl.BlockSpec((1, c1), lambda b, j: (0, 0)),
            pl.BlockSpec((c2, c1), lambda b, j: (0, 0)),
            pl.BlockSpec((1, c2), lambda b, j: (0, 0)),
        ],
        out_specs=[
            pl.BlockSpec((3, sch * NSAMPLE, c2), lambda b, j: (b, j, 0)),
            pl.BlockSpec((1, c2), lambda b, j: (0, 0)),
        ],
        out_shape=[
            jax.ShapeDtypeStruct((b3, nrows, c2), jnp.float32),
            jax.ShapeDtypeStruct((1, c2), jnp.float32),
        ],
    )(x1, u1, ss1, W2, b2.reshape(1, c2))


def _stats3_body(y2_ref, ss2_ref, w3_ref, b3_ref, ss3_ref, *, cnt2):
    b = pl.program_id(0)
    j = pl.program_id(1)

    @pl.when(jnp.logical_and(b == 0, j == 0))
    def _():
        ss3_ref[...] = jnp.zeros_like(ss3_ref)

    s2 = lax.rsqrt(ss2_ref[0] / cnt2 + 1e-5)
    y2 = y2_ref[...]
    _, rows, c2 = y2.shape
    sch = rows // NSAMPLE
    y2 = y2.reshape(3, sch, NSAMPLE, c2) * s2[None, None, None, :]
    z2 = _qrelu_triple(y2).reshape(3, rows, c2)
    w3 = w3_ref[...]
    b3v = b3_ref[...]
    for c in range(3):
        y3 = lax.dot_general(z2[c], w3, (((1,), (1,)), ((), ())),
                             preferred_element_type=jnp.float32) + b3v
        ss3_ref[...] += jnp.sum(y3 * y3, axis=0).reshape(1, -1)


def _stats3(y2, ss2, W3, b3w, sch, cnt2):
    b3, nrows, c2 = y2.shape
    c3 = W3.shape[0]
    bsz = b3 // 3
    jgrid = NPOINT // sch
    return pl.pallas_call(
        functools.partial(_stats3_body, cnt2=cnt2),
        grid=(bsz, jgrid),
        in_specs=[
            pl.BlockSpec((3, sch * NSAMPLE, c2), lambda b, j: (b, j, 0)),
            pl.BlockSpec((1, c2), lambda b, j: (0, 0)),
            pl.BlockSpec((c3, c2), lambda b, j: (0, 0)),
            pl.BlockSpec((1, c3), lambda b, j: (0, 0)),
        ],
        out_specs=pl.BlockSpec((1, c3), lambda b, j: (0, 0)),
        out_shape=jax.ShapeDtypeStruct((1, c3), jnp.float32),
    )(y2, ss2, W3, b3w.reshape(1, c3))


def _final_body(y2_ref, ss2_ref, ss3_ref, w3_ref, b3_ref, out_ref, *,
                cnt2, cnt3):
    s2 = lax.rsqrt(ss2_ref[0] / cnt2 + 1e-5)
    s3 = lax.rsqrt(ss3_ref[0] / cnt3 + 1e-5)
    y2 = y2_ref[...]
    _, rows, c2 = y2.shape
    sch = rows // NSAMPLE
    y2 = y2.reshape(3, sch, NSAMPLE, c2) * s2[None, None, None, :]
    z2 = _qrelu_triple(y2).reshape(3, rows, c2)
    w3 = w3_ref[...]
    b3v = b3_ref[...]
    c3 = w3.shape[0]
    y3l = []
    for c in range(3):
        y3 = lax.dot_general(z2[c], w3, (((1,), (1,)), ((), ())),
                             preferred_element_type=jnp.float32) + b3v
        y3l.append(y3.reshape(sch, NSAMPLE, c3))
    y3 = jnp.stack(y3l, axis=0) * s3[None, None, None, :]
    z3 = _qrelu_triple(y3)
    out_ref[...] = jnp.max(z3, axis=2)


def _final(y2, ss2, ss3, W3, b3w, sch, cnt2, cnt3):
    b3, nrows, c2 = y2.shape
    c3 = W3.shape[0]
    bsz = b3 // 3
    jgrid = NPOINT // sch
    return pl.pallas_call(
        functools.partial(_final_body, cnt2=cnt2, cnt3=cnt3),
        grid=(bsz, jgrid),
        in_specs=[
            pl.BlockSpec((3, sch * NSAMPLE, c2), lambda b, j: (b, j, 0)),
            pl.BlockSpec((1, c2), lambda b, j: (0, 0)),
            pl.BlockSpec((1, c3), lambda b, j: (0, 0)),
            pl.BlockSpec((c3, c2), lambda b, j: (0, 0)),
            pl.BlockSpec((1, c3), lambda b, j: (0, 0)),
        ],
        out_specs=pl.BlockSpec((3, sch, c3), lambda b, j: (b, j, 0)),
        out_shape=jax.ShapeDtypeStruct((b3, NPOINT, c3), jnp.float32),
    )(y2, ss2, ss3, W3, b3w.reshape(1, c3))


def kernel(xyz, points, W1, b1, W2, b2, W3, b3):
    bsz, _, n = xyz.shape
    b3n, d, _ = points.shape
    c1 = W1.shape[0]

    cent, nxyz = _fps(xyz)

    g1, u1 = _g1_u1(xyz, points, nxyz, W1, b1)

    flags = _flags(jnp.transpose(nxyz, (0, 2, 1)),
                   jnp.transpose(xyz, (0, 2, 1)))
    idx = _select_sc(flags)

    offs = (jnp.arange(b3n, dtype=jnp.int32) * n).reshape(b3n, 1)
    idxb = jnp.broadcast_to(
        idx.reshape(bsz, 1, NPOINT * NSAMPLE), (bsz, 3, NPOINT * NSAMPLE)
    ).reshape(b3n, NPOINT * NSAMPLE) + offs
    x1 = _gather_sc(g1.reshape(b3n * n, c1), idxb.reshape(-1))
    x1 = x1.reshape(b3n, NPOINT * NSAMPLE, c1)

    sch = 64
    cnt = float(b3n * NPOINT * NSAMPLE)
    ss1 = _stats1(x1, u1, sch)
    y2, ss2 = _layer2(x1, u1, ss1, W2, b2, sch, cnt)
    ss3 = _stats3(y2, ss2, W3, b3, sch, cnt)
    new_points = _final(y2, ss2, ss3, W3, b3, sch, cnt, cnt)

    return nxyz, jnp.transpose(new_points, (0, 2, 1))


# speedup vs baseline: 13.2523x; 1.1858x over previous
import functools

import jax
import jax.numpy as jnp
import numpy as np
from jax import lax
from jax.experimental import pallas as pl
from jax.experimental.pallas import tpu as pltpu
from jax.experimental.pallas import tpu_sc as plsc

NPOINT = 512
RADIUS = 0.2
NSAMPLE = 64
R2 = float(RADIUS) ** 2

SC_CORES = 2
SC_SUBCORES = 16
SC_WORKERS = SC_CORES * SC_SUBCORES
SC_LANES = 16


def _fps_body(xyz_ref, cent_ref, nx_ref):
    x = xyz_ref[...]
    bsz, _, n = x.shape
    x0 = x[:, 0, :]
    x1 = x[:, 1, :]
    x2 = x[:, 2, :]
    m0 = jnp.mean(x0, axis=1, keepdims=True)
    m1 = jnp.mean(x1, axis=1, keepdims=True)
    m2 = jnp.mean(x2, axis=1, keepdims=True)
    dist0 = (x0 - m0) ** 2 + (x1 - m1) ** 2 + (x2 - m2) ** 2
    distance = jnp.minimum(jnp.full((bsz, n), 1e10, jnp.float32), dist0)
    lane = lax.broadcasted_iota(jnp.int32, (bsz, n), 1)
    scol = lax.broadcasted_iota(jnp.int32, (bsz, NPOINT), 1)

    def argmax_rows(d):
        mv = jnp.max(d, axis=1, keepdims=True)
        return jnp.min(jnp.where(d == mv, lane, n), axis=1, keepdims=True)

    far = argmax_rows(distance)
    cent = jnp.zeros((bsz, NPOINT), jnp.int32)
    nx0 = jnp.zeros((bsz, NPOINT), jnp.float32)
    nx1 = jnp.zeros((bsz, NPOINT), jnp.float32)
    nx2 = jnp.zeros((bsz, NPOINT), jnp.float32)

    def body(i, carry):
        distance, far, cent, nx0, nx1, nx2 = carry
        oh = lane == far
        c0 = jnp.sum(jnp.where(oh, x0, 0.0), axis=1, keepdims=True)
        c1 = jnp.sum(jnp.where(oh, x1, 0.0), axis=1, keepdims=True)
        c2 = jnp.sum(jnp.where(oh, x2, 0.0), axis=1, keepdims=True)
        sel = scol == i
        cent = jnp.where(sel, far, cent)
        nx0 = jnp.where(sel, c0, nx0)
        nx1 = jnp.where(sel, c1, nx1)
        nx2 = jnp.where(sel, c2, nx2)
        d = (x0 - c0) ** 2 + (x1 - c1) ** 2 + (x2 - c2) ** 2
        distance = jnp.minimum(distance, d)
        far = argmax_rows(distance)
        return (distance, far, cent, nx0, nx1, nx2)

    carry = (distance, far, cent, nx0, nx1, nx2)
    _, _, cent, nx0, nx1, nx2 = lax.fori_loop(0, NPOINT, body, carry)
    cent_ref[...] = cent
    nx_ref[:, 0, :] = nx0
    nx_ref[:, 1, :] = nx1
    nx_ref[:, 2, :] = nx2


def _fps(xyz):
    bsz = xyz.shape[0]
    return pl.pallas_call(
        _fps_body,
        out_shape=[
            jax.ShapeDtypeStruct((bsz, NPOINT), jnp.int32),
            jax.ShapeDtypeStruct((bsz, 3, NPOINT), jnp.float32),
        ],
    )(xyz)


def _g1_body(xyz_ref, pts_ref, nx_ref, w1_ref, b1_ref, g1_ref, u1_ref):
    pts = pts_ref[0]
    xyzrow = xyz_ref[0, 0]
    nxrow = nx_ref[0, 0]
    w1 = w1_ref[...]
    b1 = b1_ref[...]
    w01 = w1[:, 0] + w1[:, 1]
    w1p = w1[:, 2:]
    g = lax.dot_general(pts, w1p, (((0,), (1,)), ((), ())),
                        preferred_element_type=jnp.float32)
    g = g + xyzrow[:, None] * w01[None, :]
    g1_ref[0] = g
    u1_ref[0] = b1 - nxrow[:, None] * w1[:, 0][None, :]


def _g1_u1(xyz, points, nxyz, W1, b1):
    b3, d, n = points.shape
    c1 = W1.shape[0]
    return pl.pallas_call(
        _g1_body,
        grid=(b3,),
        in_specs=[
            pl.BlockSpec((1, 1, n), lambda i: (i, 0, 0)),
            pl.BlockSpec((1, d, n), lambda i: (i, 0, 0)),
            pl.BlockSpec((1, 1, NPOINT), lambda i: (i, 0, 0)),
            pl.BlockSpec((c1, d + 2), lambda i: (0, 0)),
            pl.BlockSpec((1, c1), lambda i: (0, 0)),
        ],
        out_specs=[
            pl.BlockSpec((1, n, c1), lambda i: (i, 0, 0)),
            pl.BlockSpec((1, NPOINT, c1), lambda i: (i, 0, 0)),
        ],
        out_shape=[
            jax.ShapeDtypeStruct((b3, n, c1), jnp.float32),
            jax.ShapeDtypeStruct((b3, NPOINT, c1), jnp.float32),
        ],
    )(xyz.reshape(b3, 1, n), points, nxyz.reshape(b3, 1, NPOINT),
      W1, b1.reshape(1, c1))


def _flags_body(nx_ref, xt_ref, f_ref):
    src = nx_ref[0]
    dst = xt_ref[0]
    mm = lax.dot_general(src, dst, (((1,), (1,)), ((), ())))
    dist = -2.0 * mm
    dist = dist + jnp.sum(src ** 2, -1)[:, None]
    dist = dist + jnp.sum(dst ** 2, -1)[None, :]
    f_ref[0] = (dist <= R2).astype(jnp.int32)


def _flags(nx_t, xyz_t):
    bsz, n, _ = xyz_t.shape
    return pl.pallas_call(
        _flags_body,
        grid=(bsz,),
        in_specs=[pl.BlockSpec((1, NPOINT, 3), lambda i: (i, 0, 0)),
                  pl.BlockSpec((1, n, 3), lambda i: (i, 0, 0))],
        out_specs=pl.BlockSpec((1, NPOINT, n), lambda i: (i, 0, 0)),
        out_shape=jax.ShapeDtypeStruct((bsz, NPOINT, n), jnp.int32),
    )(nx_t, xyz_t)


def _select_sc(flags):
    rows = flags.shape[0] * flags.shape[1]
    n = flags.shape[2]
    flags = flags.reshape(rows, n)
    rows_per_w = rows // SC_WORKERS
    slab_rows = 16
    nslabs = rows_per_w // slab_rows
    nchunks = n // SC_LANES
    mesh = plsc.VectorSubcoreMesh(core_axis_name="c", subcore_axis_name="s",
                                  num_cores=SC_CORES, num_subcores=SC_SUBCORES)

    @functools.partial(
        pl.kernel,
        out_type=jax.ShapeDtypeStruct((rows, NSAMPLE), jnp.int32),
        mesh=mesh,
        compiler_params=pltpu.CompilerParams(needs_layout_passes=False),
        scratch_types=[
            pltpu.VMEM((slab_rows, n), jnp.int32),
            pltpu.VMEM((rows_per_w, NSAMPLE), jnp.int32),
        ],
    )
    def sel(f_hbm, idx_hbm, slabv, bufv):
        wid = lax.axis_index("s") * SC_CORES + lax.axis_index("c")
        lanes = lax.broadcasted_iota(jnp.int32, (SC_LANES,), 0)
        zeros = jnp.zeros((SC_LANES,), jnp.int32)

        def slab_body(sl, _):
            pltpu.sync_copy(
                f_hbm.at[pl.ds(wid * rows_per_w + sl * slab_rows, slab_rows)],
                slabv)

            def row_body(j2, _):
                rowv = jnp.full((SC_LANES,), j2, jnp.int32)
                browv = sl * slab_rows + rowv
                unroll = 8
                nsup = nchunks // unroll

                def cond(carry):
                    sup, cntv = carry
                    return jnp.logical_and(sup < nsup,
                                           jnp.max(cntv) < NSAMPLE)

                def step(carry):
                    sup, cntv = carry
                    for k in range(unroll):
                        col = (sup * unroll + k) * SC_LANES + lanes
                        mi = plsc.load_gather(slabv, [rowv, col])
                        m = mi == 1
                        incl = plsc.cumsum(mi)
                        pos = cntv + incl - mi
                        mstore = jnp.logical_and(m, pos < NSAMPLE)
                        plsc.store_scatter(bufv, [browv, pos], col,
                                           mask=mstore)
                        cntv = cntv + plsc.all_reduce_population_count(m)
                    return (sup + 1, cntv)

                _, cntv = lax.while_loop(
                    cond, step,
                    (jnp.int32(0), jnp.zeros((SC_LANES,), jnp.int32)))
                cnt = jnp.max(cntv)

                firstv = plsc.load_gather(bufv, [browv, zeros])
                for mch in range(NSAMPLE // SC_LANES):
                    slots = mch * SC_LANES + lanes
                    cur = plsc.load_gather(bufv, [browv, slots])
                    plsc.store_scatter(bufv, [browv, slots],
                                       jnp.where(slots < cnt, cur, firstv))
                return 0

            lax.fori_loop(0, slab_rows, row_body, 0)
            return 0

        lax.fori_loop(0, nslabs, slab_body, 0)
        pltpu.sync_copy(bufv, idx_hbm.at[pl.ds(wid * rows_per_w, rows_per_w)])

    return sel(flags)


def _gather_sc(g1flat, idxflat):
    total, c1 = g1flat.shape[0], g1flat.shape[1]
    nrows = idxflat.shape[0]
    rows_per_w = nrows // SC_WORKERS
    chunk = 128
    inner = 8
    slab = chunk * inner
    outer = rows_per_w // slab
    mesh = plsc.VectorSubcoreMesh(core_axis_name="c", subcore_axis_name="s",
                                  num_cores=SC_CORES, num_subcores=SC_SUBCORES)

    @functools.partial(
        pl.kernel,
        out_type=jax.ShapeDtypeStruct((nrows, c1), jnp.float32),
        mesh=mesh,
        compiler_params=pltpu.CompilerParams(needs_layout_passes=False,
                                             use_tc_tiling_on_sc=False),
        scratch_types=[
            pltpu.VMEM((rows_per_w,), jnp.int32),
            pltpu.VMEM((slab, c1), jnp.float32),
            pltpu.SemaphoreType.DMA,
        ],
    )
    def gat(tab_hbm, idx_hbm, out_hbm, idxv, rowsv, sem):
        wid = lax.axis_index("s") * SC_CORES + lax.axis_index("c")
        base = wid * rows_per_w
        pltpu.sync_copy(idx_hbm.at[pl.ds(base, rows_per_w)], idxv)

        def outer_body(o, _):
            handles = []
            for k in range(inner):
                src = tab_hbm.at[idxv.at[pl.ds(o * slab + k * chunk, chunk)]]
                dst = rowsv.at[pl.ds(k * chunk, chunk)]
                handles.append(pltpu.async_copy(src, dst, sem))
            for h in handles:
                h.wait()
            pltpu.sync_copy(rowsv, out_hbm.at[pl.ds(base + o * slab, slab)])
            return 0

        lax.fori_loop(0, outer, outer_body, 0)

    return gat(g1flat, idxflat)


def _qrelu_triple(y):
    q = jnp.sqrt(y[0] * y[0] + y[1] * y[1] + y[2] * y[2])
    coef = q / jnp.maximum(q, 1.0)
    return y * coef[None]


def _stats1_body(x_ref, u_ref, ss_ref):
    i = pl.program_id(0)
    j = pl.program_id(1)

    @pl.when(jnp.logical_and(i == 0, j == 0))
    def _():
        ss_ref[...] = jnp.zeros_like(ss_ref)

    x = x_ref[0]
    u = u_ref[0]
    sch, c = u.shape
    y = x.reshape(sch, NSAMPLE, c) + u[:, None, :]
    ss_ref[...] += jnp.sum(y * y, axis=(0, 1)).reshape(1, c)


def _stats1(x1, u1, sch):
    b3, nrows, c1 = x1.shape
    jgrid = NPOINT // sch
    return pl.pallas_call(
        _stats1_body,
        grid=(b3, jgrid),
        in_specs=[
            pl.BlockSpec((1, sch * NSAMPLE, c1), lambda i, j: (i, j, 0)),
            pl.BlockSpec((1, sch, c1), lambda i, j: (i, j, 0)),
        ],
        out_specs=pl.BlockSpec((1, c1), lambda i, j: (0, 0)),
        out_shape=jax.ShapeDtypeStruct((1, c1), jnp.float32),
    )(x1, u1)


def _layer2_body(x_ref, u_ref, ss1_ref, w2_ref, b2_ref, y2_ref, ss2_ref, *,
                 cnt1):
    b = pl.program_id(0)
    j = pl.program_id(1)

    @pl.when(jnp.logical_and(b == 0, j == 0))
    def _():
        ss2_ref[...] = jnp.zeros_like(ss2_ref)

    s1 = lax.rsqrt(ss1_ref[0] / cnt1 + 1e-5)
    x = x_ref[...]
    u = u_ref[...]
    _, sch, c1 = u.shape
    y1 = x.reshape(3, sch, NSAMPLE, c1) + u[:, :, None, :]
    y1 = y1 * s1[None, None, None, :]
    z1 = _qrelu_triple(y1).reshape(3, sch * NSAMPLE, c1)
    w2 = w2_ref[...]
    b2 = b2_ref[...]
    for c in range(3):
        y2 = lax.dot_general(z1[c], w2, (((1,), (1,)), ((), ())),
                             preferred_element_type=jnp.float32) + b2
        y2_ref[c] = y2
        ss2_ref[...] += jnp.sum(y2 * y2, axis=0).reshape(1, -1)


def _layer2(x1, u1, ss1, W2, b2, sch, cnt1):
    b3, nrows, c1 = x1.shape
    c2 = W2.shape[0]
    bsz = b3 // 3
    jgrid = NPOINT // sch
    return pl.pallas_call(
        functools.partial(_layer2_body, cnt1=cnt1),
        grid=(bsz, jgrid),
        in_specs=[
            pl.BlockSpec((3, sch * NSAMPLE, c1), lambda b, j: (b, j, 0)),
            pl.BlockSpec((3, sch, c1), lambda b, j: (b, j, 0)),
            pl.BlockSpec((1, c1), lambda b, j: (0, 0)),
            pl.BlockSpec((c2, c1), lambda b, j: (0, 0)),
            pl.BlockSpec((1, c2), lambda b, j: (0, 0)),
        ],
        out_specs=[
            pl.BlockSpec((3, sch * NSAMPLE, c2), lambda b, j: (b, j, 0)),
            pl.BlockSpec((1, c2), lambda b, j: (0, 0)),
        ],
        out_shape=[
            jax.ShapeDtypeStruct((b3, nrows, c2), jnp.float32),
            jax.ShapeDtypeStruct((1, c2), jnp.float32),
        ],
    )(x1, u1, ss1, W2, b2.reshape(1, c2))


def _stats3_body(y2_ref, ss2_ref, w3_ref, b3_ref, ss3_ref, *, cnt2):
    b = pl.program_id(0)
    j = pl.program_id(1)

    @pl.when(jnp.logical_and(b == 0, j == 0))
    def _():
        ss3_ref[...] = jnp.zeros_like(ss3_ref)

    s2 = lax.rsqrt(ss2_ref[0] / cnt2 + 1e-5)
    y2 = y2_ref[...]
    _, rows, c2 = y2.shape
    sch = rows // NSAMPLE
    y2 = y2.reshape(3, sch, NSAMPLE, c2) * s2[None, None, None, :]
    z2 = _qrelu_triple(y2).reshape(3, rows, c2)
    w3 = w3_ref[...]
    b3v = b3_ref[...]
    for c in range(3):
        y3 = lax.dot_general(z2[c], w3, (((1,), (1,)), ((), ())),
                             preferred_element_type=jnp.float32) + b3v
        ss3_ref[...] += jnp.sum(y3 * y3, axis=0).reshape(1, -1)


def _stats3(y2, ss2, W3, b3w, sch, cnt2):
    b3, nrows, c2 = y2.shape
    c3 = W3.shape[0]
    bsz = b3 // 3
    jgrid = NPOINT // sch
    return pl.pallas_call(
        functools.partial(_stats3_body, cnt2=cnt2),
        grid=(bsz, jgrid),
        in_specs=[
            pl.BlockSpec((3, sch * NSAMPLE, c2), lambda b, j: (b, j, 0)),
            pl.BlockSpec((1, c2), lambda b, j: (0, 0)),
            pl.BlockSpec((c3, c2), lambda b, j: (0, 0)),
            pl.BlockSpec((1, c3), lambda b, j: (0, 0)),
        ],
        out_specs=pl.BlockSpec((1, c3), lambda b, j: (0, 0)),
        out_shape=jax.ShapeDtypeStruct((1, c3), jnp.float32),
    )(y2, ss2, W3, b3w.reshape(1, c3))


def _final_body(y2_ref, ss2_ref, ss3_ref, w3_ref, b3_ref, out_ref, *,
                cnt2, cnt3):
    s2 = lax.rsqrt(ss2_ref[0] / cnt2 + 1e-5)
    s3 = lax.rsqrt(ss3_ref[0] / cnt3 + 1e-5)
    y2 = y2_ref[...]
    _, rows, c2 = y2.shape
    sch = rows // NSAMPLE
    y2 = y2.reshape(3, sch, NSAMPLE, c2) * s2[None, None, None, :]
    z2 = _qrelu_triple(y2).reshape(3, rows, c2)
    w3 = w3_ref[...]
    b3v = b3_ref[...]
    c3 = w3.shape[0]
    y3l = []
    for c in range(3):
        y3 = lax.dot_general(z2[c], w3, (((1,), (1,)), ((), ())),
                             preferred_element_type=jnp.float32) + b3v
        y3l.append(y3.reshape(sch, NSAMPLE, c3))
    y3 = jnp.stack(y3l, axis=0) * s3[None, None, None, :]
    z3 = _qrelu_triple(y3)
    out_ref[...] = jnp.max(z3, axis=2)


def _final(y2, ss2, ss3, W3, b3w, sch, cnt2, cnt3):
    b3, nrows, c2 = y2.shape
    c3 = W3.shape[0]
    bsz = b3 // 3
    jgrid = NPOINT // sch
    return pl.pallas_call(
        functools.partial(_final_body, cnt2=cnt2, cnt3=cnt3),
        grid=(bsz, jgrid),
        in_specs=[
            pl.BlockSpec((3, sch * NSAMPLE, c2), lambda b, j: (b, j, 0)),
            pl.BlockSpec((1, c2), lambda b, j: (0, 0)),
            pl.BlockSpec((1, c3), lambda b, j: (0, 0)),
            pl.BlockSpec((c3, c2), lambda b, j: (0, 0)),
            pl.BlockSpec((1, c3), lambda b, j: (0, 0)),
        ],
        out_specs=pl.BlockSpec((3, sch, c3), lambda b, j: (b, j, 0)),
        out_shape=jax.ShapeDtypeStruct((b3, NPOINT, c3), jnp.float32),
    )(y2, ss2, ss3, W3, b3w.reshape(1, c3))


def kernel(xyz, points, W1, b1, W2, b2, W3, b3):
    bsz, _, n = xyz.shape
    b3n, d, _ = points.shape
    c1 = W1.shape[0]

    cent, nxyz = _fps(xyz)

    g1, u1 = _g1_u1(xyz, points, nxyz, W1, b1)

    flags = _flags(jnp.transpose(nxyz, (0, 2, 1)),
                   jnp.transpose(xyz, (0, 2, 1)))
    idx = _select_sc(flags)

    offs = (jnp.arange(b3n, dtype=jnp.int32) * n).reshape(b3n, 1)
    idxb = jnp.broadcast_to(
        idx.reshape(bsz, 1, NPOINT * NSAMPLE), (bsz, 3, NPOINT * NSAMPLE)
    ).reshape(b3n, NPOINT * NSAMPLE) + offs
    x1 = _gather_sc(g1.reshape(b3n * n, c1), idxb.reshape(-1))
    x1 = x1.reshape(b3n, NPOINT * NSAMPLE, c1)

    sch = 64
    cnt = float(b3n * NPOINT * NSAMPLE)
    ss1 = _stats1(x1, u1, sch)
    y2, ss2 = _layer2(x1, u1, ss1, W2, b2, sch, cnt)
    ss3 = _stats3(y2, ss2, W3, b3, sch, cnt)
    new_points = _final(y2, ss2, ss3, W3, b3, sch, cnt, cnt)

    return nxyz, jnp.transpose(new_points, (0, 2, 1))


# speedup vs baseline: 13.3603x; 1.0081x over previous
import functools

import jax
import jax.numpy as jnp
import numpy as np
from jax import lax
from jax.experimental import pallas as pl
from jax.experimental.pallas import tpu as pltpu
from jax.experimental.pallas import tpu_sc as plsc

NPOINT = 512
RADIUS = 0.2
NSAMPLE = 64
R2 = float(RADIUS) ** 2

SC_CORES = 2
SC_SUBCORES = 16
SC_WORKERS = SC_CORES * SC_SUBCORES
SC_LANES = 16


def _fps_body(xyz_ref, cent_ref, nx_ref):
    x = xyz_ref[...]
    bsz, _, n = x.shape
    x0 = x[:, 0, :]
    x1 = x[:, 1, :]
    x2 = x[:, 2, :]
    m0 = jnp.mean(x0, axis=1, keepdims=True)
    m1 = jnp.mean(x1, axis=1, keepdims=True)
    m2 = jnp.mean(x2, axis=1, keepdims=True)
    dist0 = (x0 - m0) ** 2 + (x1 - m1) ** 2 + (x2 - m2) ** 2
    distance = jnp.minimum(jnp.full((bsz, n), 1e10, jnp.float32), dist0)
    lane = lax.broadcasted_iota(jnp.int32, (bsz, n), 1)
    scol = lax.broadcasted_iota(jnp.int32, (bsz, NPOINT), 1)

    def argmax_rows(d):
        mv = jnp.max(d, axis=1, keepdims=True)
        return jnp.min(jnp.where(d == mv, lane, n), axis=1, keepdims=True)

    far = argmax_rows(distance)
    cent = jnp.zeros((bsz, NPOINT), jnp.int32)
    nx0 = jnp.zeros((bsz, NPOINT), jnp.float32)
    nx1 = jnp.zeros((bsz, NPOINT), jnp.float32)
    nx2 = jnp.zeros((bsz, NPOINT), jnp.float32)

    def body(i, carry):
        distance, far, cent, nx0, nx1, nx2 = carry
        oh = lane == far
        c0 = jnp.sum(jnp.where(oh, x0, 0.0), axis=1, keepdims=True)
        c1 = jnp.sum(jnp.where(oh, x1, 0.0), axis=1, keepdims=True)
        c2 = jnp.sum(jnp.where(oh, x2, 0.0), axis=1, keepdims=True)
        sel = scol == i
        cent = jnp.where(sel, far, cent)
        nx0 = jnp.where(sel, c0, nx0)
        nx1 = jnp.where(sel, c1, nx1)
        nx2 = jnp.where(sel, c2, nx2)
        d = (x0 - c0) ** 2 + (x1 - c1) ** 2 + (x2 - c2) ** 2
        distance = jnp.minimum(distance, d)
        far = argmax_rows(distance)
        return (distance, far, cent, nx0, nx1, nx2)

    carry = (distance, far, cent, nx0, nx1, nx2)
    _, _, cent, nx0, nx1, nx2 = lax.fori_loop(0, NPOINT, body, carry)
    cent_ref[...] = cent
    nx_ref[:, 0, :] = nx0
    nx_ref[:, 1, :] = nx1
    nx_ref[:, 2, :] = nx2


def _fps(xyz):
    bsz = xyz.shape[0]
    return pl.pallas_call(
        _fps_body,
        out_shape=[
            jax.ShapeDtypeStruct((bsz, NPOINT), jnp.int32),
            jax.ShapeDtypeStruct((bsz, 3, NPOINT), jnp.float32),
        ],
    )(xyz)


def _g1_body(xyz_ref, pts_ref, nx_ref, w1_ref, b1_ref, g1_ref, u1_ref):
    pts = pts_ref[0]
    xyzrow = xyz_ref[0, 0]
    nxrow = nx_ref[0, 0]
    w1 = w1_ref[...]
    b1 = b1_ref[...]
    w01 = w1[:, 0] + w1[:, 1]
    w1p = w1[:, 2:]
    g = lax.dot_general(pts, w1p, (((0,), (1,)), ((), ())),
                        preferred_element_type=jnp.float32)
    g = g + xyzrow[:, None] * w01[None, :]
    g1_ref[0] = g
    u1_ref[0] = b1 - nxrow[:, None] * w1[:, 0][None, :]


def _g1_u1(xyz, points, nxyz, W1, b1):
    b3, d, n = points.shape
    c1 = W1.shape[0]
    return pl.pallas_call(
        _g1_body,
        grid=(b3,),
        in_specs=[
            pl.BlockSpec((1, 1, n), lambda i: (i, 0, 0)),
            pl.BlockSpec((1, d, n), lambda i: (i, 0, 0)),
            pl.BlockSpec((1, 1, NPOINT), lambda i: (i, 0, 0)),
            pl.BlockSpec((c1, d + 2), lambda i: (0, 0)),
            pl.BlockSpec((1, c1), lambda i: (0, 0)),
        ],
        out_specs=[
            pl.BlockSpec((1, n, c1), lambda i: (i, 0, 0)),
            pl.BlockSpec((1, NPOINT, c1), lambda i: (i, 0, 0)),
        ],
        out_shape=[
            jax.ShapeDtypeStruct((b3, n, c1), jnp.float32),
            jax.ShapeDtypeStruct((b3, NPOINT, c1), jnp.float32),
        ],
    )(xyz.reshape(b3, 1, n), points, nxyz.reshape(b3, 1, NPOINT),
      W1, b1.reshape(1, c1))


def _flags_body(nx_ref, xt_ref, f_ref):
    src = nx_ref[0]
    dst = xt_ref[0]
    mm = lax.dot_general(src, dst, (((1,), (1,)), ((), ())))
    dist = -2.0 * mm
    dist = dist + jnp.sum(src ** 2, -1)[:, None]
    dist = dist + jnp.sum(dst ** 2, -1)[None, :]
    f_ref[0] = (dist <= R2).astype(jnp.int32)


def _flags(nx_t, xyz_t):
    bsz, n, _ = xyz_t.shape
    return pl.pallas_call(
        _flags_body,
        grid=(bsz,),
        in_specs=[pl.BlockSpec((1, NPOINT, 3), lambda i: (i, 0, 0)),
                  pl.BlockSpec((1, n, 3), lambda i: (i, 0, 0))],
        out_specs=pl.BlockSpec((1, NPOINT, n), lambda i: (i, 0, 0)),
        out_shape=jax.ShapeDtypeStruct((bsz, NPOINT, n), jnp.int32),
    )(nx_t, xyz_t)


def _select_sc(flags):
    rows = flags.shape[0] * flags.shape[1]
    n = flags.shape[2]
    flags = flags.reshape(rows, n)
    rows_per_w = rows // SC_WORKERS
    slab_rows = 16
    nslabs = rows_per_w // slab_rows
    nchunks = n // SC_LANES
    mesh = plsc.VectorSubcoreMesh(core_axis_name="c", subcore_axis_name="s",
                                  num_cores=SC_CORES, num_subcores=SC_SUBCORES)

    @functools.partial(
        pl.kernel,
        out_type=jax.ShapeDtypeStruct((rows, NSAMPLE), jnp.int32),
        mesh=mesh,
        compiler_params=pltpu.CompilerParams(needs_layout_passes=False),
        scratch_types=[
            pltpu.VMEM((slab_rows, n), jnp.int32),
            pltpu.VMEM((rows_per_w, NSAMPLE), jnp.int32),
        ],
    )
    def sel(f_hbm, idx_hbm, slabv, bufv):
        wid = lax.axis_index("s") * SC_CORES + lax.axis_index("c")
        lanes = lax.broadcasted_iota(jnp.int32, (SC_LANES,), 0)
        zeros = jnp.zeros((SC_LANES,), jnp.int32)

        def slab_body(sl, _):
            pltpu.sync_copy(
                f_hbm.at[pl.ds(wid * rows_per_w + sl * slab_rows, slab_rows)],
                slabv)

            def row_body(j2, _):
                rowv = jnp.full((SC_LANES,), j2, jnp.int32)
                browv = sl * slab_rows + rowv
                unroll = 8
                nsup = nchunks // unroll

                def cond(carry):
                    sup, cntv = carry
                    return jnp.logical_and(sup < nsup,
                                           jnp.max(cntv) < NSAMPLE)

                def step(carry):
                    sup, cntv = carry
                    for k in range(unroll):
                        col = (sup * unroll + k) * SC_LANES + lanes
                        mi = plsc.load_gather(slabv, [rowv, col])
                        m = mi == 1
                        incl = plsc.cumsum(mi)
                        pos = cntv + incl - mi
                        mstore = jnp.logical_and(m, pos < NSAMPLE)
                        plsc.store_scatter(bufv, [browv, pos], col,
                                           mask=mstore)
                        cntv = cntv + plsc.all_reduce_population_count(m)
                    return (sup + 1, cntv)

                _, cntv = lax.while_loop(
                    cond, step,
                    (jnp.int32(0), jnp.zeros((SC_LANES,), jnp.int32)))
                cnt = jnp.max(cntv)

                firstv = plsc.load_gather(bufv, [browv, zeros])
                for mch in range(NSAMPLE // SC_LANES):
                    slots = mch * SC_LANES + lanes
                    cur = plsc.load_gather(bufv, [browv, slots])
                    plsc.store_scatter(bufv, [browv, slots],
                                       jnp.where(slots < cnt, cur, firstv))
                return 0

            lax.fori_loop(0, slab_rows, row_body, 0)
            return 0

        lax.fori_loop(0, nslabs, slab_body, 0)
        pltpu.sync_copy(bufv, idx_hbm.at[pl.ds(wid * rows_per_w, rows_per_w)])

    return sel(flags)


def _gather_sc(g1flat, idxflat):
    total, c1 = g1flat.shape[0], g1flat.shape[1]
    nrows = idxflat.shape[0]
    rows_per_w = nrows // SC_WORKERS
    chunk = 128
    inner = 8
    slab = chunk * inner
    outer = rows_per_w // slab
    mesh = plsc.VectorSubcoreMesh(core_axis_name="c", subcore_axis_name="s",
                                  num_cores=SC_CORES, num_subcores=SC_SUBCORES)

    @functools.partial(
        pl.kernel,
        out_type=jax.ShapeDtypeStruct((nrows, c1), jnp.float32),
        mesh=mesh,
        compiler_params=pltpu.CompilerParams(needs_layout_passes=False,
                                             use_tc_tiling_on_sc=False),
        scratch_types=[
            pltpu.VMEM((rows_per_w,), jnp.int32),
            pltpu.VMEM((slab, c1), jnp.float32),
            pltpu.SemaphoreType.DMA,
        ],
    )
    def gat(tab_hbm, idx_hbm, out_hbm, idxv, rowsv, sem):
        wid = lax.axis_index("s") * SC_CORES + lax.axis_index("c")
        base = wid * rows_per_w
        pltpu.sync_copy(idx_hbm.at[pl.ds(base, rows_per_w)], idxv)

        def outer_body(o, _):
            handles = []
            for k in range(inner):
                src = tab_hbm.at[idxv.at[pl.ds(o * slab + k * chunk, chunk)]]
                dst = rowsv.at[pl.ds(k * chunk, chunk)]
                handles.append(pltpu.async_copy(src, dst, sem))
            for h in handles:
                h.wait()
            pltpu.sync_copy(rowsv, out_hbm.at[pl.ds(base + o * slab, slab)])
            return 0

        lax.fori_loop(0, outer, outer_body, 0)

    return gat(g1flat, idxflat)


def _qrelu_triple(y):
    q = jnp.sqrt(y[0] * y[0] + y[1] * y[1] + y[2] * y[2])
    coef = q / jnp.maximum(q, 1.0)
    return y * coef[None]


def _stats1_body(x_ref, u_ref, ss_ref):
    i = pl.program_id(0)
    j = pl.program_id(1)

    @pl.when(jnp.logical_and(i == 0, j == 0))
    def _():
        ss_ref[...] = jnp.zeros_like(ss_ref)

    x = x_ref[0]
    u = u_ref[0]
    sch, c = u.shape
    y = x.reshape(sch, NSAMPLE, c) + u[:, None, :]
    ss_ref[...] += jnp.sum(y * y, axis=(0, 1)).reshape(1, c)


def _stats1(x1, u1, sch):
    b3, nrows, c1 = x1.shape
    jgrid = NPOINT // sch
    return pl.pallas_call(
        _stats1_body,
        grid=(b3, jgrid),
        in_specs=[
            pl.BlockSpec((1, sch * NSAMPLE, c1), lambda i, j: (i, j, 0)),
            pl.BlockSpec((1, sch, c1), lambda i, j: (i, j, 0)),
        ],
        out_specs=pl.BlockSpec((1, c1), lambda i, j: (0, 0)),
        out_shape=jax.ShapeDtypeStruct((1, c1), jnp.float32),
    )(x1, u1)


def _layer2_body(x_ref, u_ref, ss1_ref, w2_ref, b2_ref, y2_ref, ss2_ref, *,
                 cnt1):
    b = pl.program_id(0)
    j = pl.program_id(1)

    @pl.when(jnp.logical_and(b == 0, j == 0))
    def _():
        ss2_ref[...] = jnp.zeros_like(ss2_ref)

    s1 = lax.rsqrt(ss1_ref[0] / cnt1 + 1e-5)
    x = x_ref[...]
    u = u_ref[...]
    _, sch, c1 = u.shape
    y1 = x.reshape(3, sch, NSAMPLE, c1) + u[:, :, None, :]
    y1 = y1 * s1[None, None, None, :]
    z1 = _qrelu_triple(y1).reshape(3, sch * NSAMPLE, c1)
    w2 = w2_ref[...]
    b2 = b2_ref[...]
    for c in range(3):
        y2 = lax.dot_general(z1[c].astype(jnp.bfloat16), w2,
                             (((1,), (1,)), ((), ())),
                             preferred_element_type=jnp.float32) + b2
        y2_ref[c] = y2.astype(jnp.bfloat16)
        ss2_ref[...] += jnp.sum(y2 * y2, axis=0).reshape(1, -1)


def _layer2(x1, u1, ss1, W2, b2, sch, cnt1):
    b3, nrows, c1 = x1.shape
    c2 = W2.shape[0]
    bsz = b3 // 3
    jgrid = NPOINT // sch
    return pl.pallas_call(
        functools.partial(_layer2_body, cnt1=cnt1),
        grid=(bsz, jgrid),
        in_specs=[
            pl.BlockSpec((3, sch * NSAMPLE, c1), lambda b, j: (b, j, 0)),
            pl.BlockSpec((3, sch, c1), lambda b, j: (b, j, 0)),
            pl.BlockSpec((1, c1), lambda b, j: (0, 0)),
            pl.BlockSpec((c2, c1), lambda b, j: (0, 0)),
            pl.BlockSpec((1, c2), lambda b, j: (0, 0)),
        ],
        out_specs=[
            pl.BlockSpec((3, sch * NSAMPLE, c2), lambda b, j: (b, j, 0)),
            pl.BlockSpec((1, c2), lambda b, j: (0, 0)),
        ],
        out_shape=[
            jax.ShapeDtypeStruct((b3, nrows, c2), jnp.bfloat16),
            jax.ShapeDtypeStruct((1, c2), jnp.float32),
        ],
    )(x1, u1, ss1, W2.astype(jnp.bfloat16), b2.reshape(1, c2))


def _stats3_body(y2_ref, ss2_ref, w3_ref, b3_ref, ss3_ref, *, cnt2):
    b = pl.program_id(0)
    j = pl.program_id(1)

    @pl.when(jnp.logical_and(b == 0, j == 0))
    def _():
        ss3_ref[...] = jnp.zeros_like(ss3_ref)

    s2 = lax.rsqrt(ss2_ref[0] / cnt2 + 1e-5)
    y2 = y2_ref[...].astype(jnp.float32)
    _, rows, c2 = y2.shape
    sch = rows // NSAMPLE
    y2 = y2.reshape(3, sch, NSAMPLE, c2) * s2[None, None, None, :]
    z2 = _qrelu_triple(y2).reshape(3, rows, c2)
    w3 = w3_ref[...]
    b3v = b3_ref[...]
    for c in range(3):
        y3 = lax.dot_general(z2[c].astype(jnp.bfloat16), w3,
                             (((1,), (1,)), ((), ())),
                             preferred_element_type=jnp.float32) + b3v
        ss3_ref[...] += jnp.sum(y3 * y3, axis=0).reshape(1, -1)


def _stats3(y2, ss2, W3, b3w, sch, cnt2):
    b3, nrows, c2 = y2.shape
    c3 = W3.shape[0]
    bsz = b3 // 3
    jgrid = NPOINT // sch
    return pl.pallas_call(
        functools.partial(_stats3_body, cnt2=cnt2),
        grid=(bsz, jgrid),
        in_specs=[
            pl.BlockSpec((3, sch * NSAMPLE, c2), lambda b, j: (b, j, 0)),
            pl.BlockSpec((1, c2), lambda b, j: (0, 0)),
            pl.BlockSpec((c3, c2), lambda b, j: (0, 0)),
            pl.BlockSpec((1, c3), lambda b, j: (0, 0)),
        ],
        out_specs=pl.BlockSpec((1, c3), lambda b, j: (0, 0)),
        out_shape=jax.ShapeDtypeStruct((1, c3), jnp.float32),
    )(y2, ss2, W3.astype(jnp.bfloat16), b3w.reshape(1, c3))


def _final_body(y2_ref, ss2_ref, ss3_ref, w3_ref, b3_ref, out_ref, *,
                cnt2, cnt3):
    s2 = lax.rsqrt(ss2_ref[0] / cnt2 + 1e-5)
    s3 = lax.rsqrt(ss3_ref[0] / cnt3 + 1e-5)
    y2 = y2_ref[...].astype(jnp.float32)
    _, rows, c2 = y2.shape
    sch = rows // NSAMPLE
    y2 = y2.reshape(3, sch, NSAMPLE, c2) * s2[None, None, None, :]
    z2 = _qrelu_triple(y2).reshape(3, rows, c2)
    w3 = w3_ref[...]
    b3v = b3_ref[...]
    c3 = w3.shape[0]
    y3l = []
    for c in range(3):
        y3 = lax.dot_general(z2[c].astype(jnp.bfloat16), w3,
                             (((1,), (1,)), ((), ())),
                             preferred_element_type=jnp.float32) + b3v
        y3l.append(y3.reshape(sch, NSAMPLE, c3))
    y3 = jnp.stack(y3l, axis=0) * s3[None, None, None, :]
    z3 = _qrelu_triple(y3)
    out_ref[...] = jnp.max(z3, axis=2)


def _final(y2, ss2, ss3, W3, b3w, sch, cnt2, cnt3):
    b3, nrows, c2 = y2.shape
    c3 = W3.shape[0]
    bsz = b3 // 3
    jgrid = NPOINT // sch
    return pl.pallas_call(
        functools.partial(_final_body, cnt2=cnt2, cnt3=cnt3),
        grid=(bsz, jgrid),
        in_specs=[
            pl.BlockSpec((3, sch * NSAMPLE, c2), lambda b, j: (b, j, 0)),
            pl.BlockSpec((1, c2), lambda b, j: (0, 0)),
            pl.BlockSpec((1, c3), lambda b, j: (0, 0)),
            pl.BlockSpec((c3, c2), lambda b, j: (0, 0)),
            pl.BlockSpec((1, c3), lambda b, j: (0, 0)),
        ],
        out_specs=pl.BlockSpec((3, sch, c3), lambda b, j: (b, j, 0)),
        out_shape=jax.ShapeDtypeStruct((b3, NPOINT, c3), jnp.float32),
    )(y2, ss2, ss3, W3.astype(jnp.bfloat16), b3w.reshape(1, c3))


def kernel(xyz, points, W1, b1, W2, b2, W3, b3):
    bsz, _, n = xyz.shape
    b3n, d, _ = points.shape
    c1 = W1.shape[0]

    cent, nxyz = _fps(xyz)

    g1, u1 = _g1_u1(xyz, points, nxyz, W1, b1)

    flags = _flags(jnp.transpose(nxyz, (0, 2, 1)),
                   jnp.transpose(xyz, (0, 2, 1)))
    idx = _select_sc(flags)

    offs = (jnp.arange(b3n, dtype=jnp.int32) * n).reshape(b3n, 1)
    idxb = jnp.broadcast_to(
        idx.reshape(bsz, 1, NPOINT * NSAMPLE), (bsz, 3, NPOINT * NSAMPLE)
    ).reshape(b3n, NPOINT * NSAMPLE) + offs
    x1 = _gather_sc(g1.reshape(b3n * n, c1), idxb.reshape(-1))
    x1 = x1.reshape(b3n, NPOINT * NSAMPLE, c1)

    sch = 64
    cnt = float(b3n * NPOINT * NSAMPLE)
    ss1 = _stats1(x1, u1, sch)
    y2, ss2 = _layer2(x1, u1, ss1, W2, b2, sch, cnt)
    ss3 = _stats3(y2, ss2, W3, b3, sch, cnt)
    new_points = _final(y2, ss2, ss3, W3, b3, sch, cnt, cnt)

    return nxyz, jnp.transpose(new_points, (0, 2, 1))


# speedup vs baseline: 13.5971x; 1.0177x over previous
import functools

import jax
import jax.numpy as jnp
import numpy as np
from jax import lax
from jax.experimental import pallas as pl
from jax.experimental.pallas import tpu as pltpu
from jax.experimental.pallas import tpu_sc as plsc

NPOINT = 512
RADIUS = 0.2
NSAMPLE = 64
R2 = float(RADIUS) ** 2

SC_CORES = 2
SC_SUBCORES = 16
SC_WORKERS = SC_CORES * SC_SUBCORES
SC_LANES = 16


def _fps_body(xyz_ref, cent_ref, nx_ref):
    x = xyz_ref[...]
    bsz, _, n = x.shape
    x0 = x[:, 0, :]
    x1 = x[:, 1, :]
    x2 = x[:, 2, :]
    m0 = jnp.mean(x0, axis=1, keepdims=True)
    m1 = jnp.mean(x1, axis=1, keepdims=True)
    m2 = jnp.mean(x2, axis=1, keepdims=True)
    dist0 = (x0 - m0) ** 2 + (x1 - m1) ** 2 + (x2 - m2) ** 2
    distance = jnp.minimum(jnp.full((bsz, n), 1e10, jnp.float32), dist0)
    lane = lax.broadcasted_iota(jnp.int32, (bsz, n), 1)
    scol = lax.broadcasted_iota(jnp.int32, (bsz, NPOINT), 1)

    def argmax_rows(d):
        mv = jnp.max(d, axis=1, keepdims=True)
        return jnp.min(jnp.where(d == mv, lane, n), axis=1, keepdims=True)

    far = argmax_rows(distance)
    cent = jnp.zeros((bsz, NPOINT), jnp.int32)
    nx0 = jnp.zeros((bsz, NPOINT), jnp.float32)
    nx1 = jnp.zeros((bsz, NPOINT), jnp.float32)
    nx2 = jnp.zeros((bsz, NPOINT), jnp.float32)

    def body(i, carry):
        distance, far, cent, nx0, nx1, nx2 = carry
        oh = lane == far
        c0 = jnp.sum(jnp.where(oh, x0, 0.0), axis=1, keepdims=True)
        c1 = jnp.sum(jnp.where(oh, x1, 0.0), axis=1, keepdims=True)
        c2 = jnp.sum(jnp.where(oh, x2, 0.0), axis=1, keepdims=True)
        sel = scol == i
        cent = jnp.where(sel, far, cent)
        nx0 = jnp.where(sel, c0, nx0)
        nx1 = jnp.where(sel, c1, nx1)
        nx2 = jnp.where(sel, c2, nx2)
        d = (x0 - c0) ** 2 + (x1 - c1) ** 2 + (x2 - c2) ** 2
        distance = jnp.minimum(distance, d)
        far = argmax_rows(distance)
        return (distance, far, cent, nx0, nx1, nx2)

    carry = (distance, far, cent, nx0, nx1, nx2)
    _, _, cent, nx0, nx1, nx2 = lax.fori_loop(0, NPOINT, body, carry)
    cent_ref[...] = cent
    nx_ref[:, 0, :] = nx0
    nx_ref[:, 1, :] = nx1
    nx_ref[:, 2, :] = nx2


def _fps(xyz):
    bsz = xyz.shape[0]
    return pl.pallas_call(
        _fps_body,
        out_shape=[
            jax.ShapeDtypeStruct((bsz, NPOINT), jnp.int32),
            jax.ShapeDtypeStruct((bsz, 3, NPOINT), jnp.float32),
        ],
    )(xyz)


def _g1_body(xyz_ref, pts_ref, nx_ref, w1_ref, b1_ref, g1_ref, u1_ref):
    pts = pts_ref[0]
    xyzrow = xyz_ref[0, 0]
    nxrow = nx_ref[0, 0]
    w1 = w1_ref[...]
    b1 = b1_ref[...]
    w01 = w1[:, 0] + w1[:, 1]
    w1p = w1[:, 2:]
    g = lax.dot_general(pts, w1p, (((0,), (1,)), ((), ())),
                        preferred_element_type=jnp.float32)
    g = g + xyzrow[:, None] * w01[None, :]
    g1_ref[0] = g
    u1_ref[0] = b1 - nxrow[:, None] * w1[:, 0][None, :]


def _g1_u1(xyz, points, nxyz, W1, b1):
    b3, d, n = points.shape
    c1 = W1.shape[0]
    return pl.pallas_call(
        _g1_body,
        grid=(b3,),
        in_specs=[
            pl.BlockSpec((1, 1, n), lambda i: (i, 0, 0)),
            pl.BlockSpec((1, d, n), lambda i: (i, 0, 0)),
            pl.BlockSpec((1, 1, NPOINT), lambda i: (i, 0, 0)),
            pl.BlockSpec((c1, d + 2), lambda i: (0, 0)),
            pl.BlockSpec((1, c1), lambda i: (0, 0)),
        ],
        out_specs=[
            pl.BlockSpec((1, n, c1), lambda i: (i, 0, 0)),
            pl.BlockSpec((1, NPOINT, c1), lambda i: (i, 0, 0)),
        ],
        out_shape=[
            jax.ShapeDtypeStruct((b3, n, c1), jnp.float32),
            jax.ShapeDtypeStruct((b3, NPOINT, c1), jnp.float32),
        ],
    )(xyz.reshape(b3, 1, n), points, nxyz.reshape(b3, 1, NPOINT),
      W1, b1.reshape(1, c1))


def _flags_body(nx_ref, xt_ref, f_ref):
    src = nx_ref[0]
    dst = xt_ref[0]
    mm = lax.dot_general(src, dst, (((1,), (1,)), ((), ())))
    dist = -2.0 * mm
    dist = dist + jnp.sum(src ** 2, -1)[:, None]
    dist = dist + jnp.sum(dst ** 2, -1)[None, :]
    f_ref[0] = (dist <= R2).astype(jnp.int32)


def _flags(nx_t, xyz_t):
    bsz, n, _ = xyz_t.shape
    return pl.pallas_call(
        _flags_body,
        grid=(bsz,),
        in_specs=[pl.BlockSpec((1, NPOINT, 3), lambda i: (i, 0, 0)),
                  pl.BlockSpec((1, n, 3), lambda i: (i, 0, 0))],
        out_specs=pl.BlockSpec((1, NPOINT, n), lambda i: (i, 0, 0)),
        out_shape=jax.ShapeDtypeStruct((bsz, NPOINT, n), jnp.int32),
    )(nx_t, xyz_t)


def _select_sc(flags):
    rows = flags.shape[0] * flags.shape[1]
    n = flags.shape[2]
    flags = flags.reshape(rows, n)
    rows_per_w = rows // SC_WORKERS
    slab_rows = 16
    nslabs = rows_per_w // slab_rows
    nchunks = n // SC_LANES
    mesh = plsc.VectorSubcoreMesh(core_axis_name="c", subcore_axis_name="s",
                                  num_cores=SC_CORES, num_subcores=SC_SUBCORES)

    @functools.partial(
        pl.kernel,
        out_type=jax.ShapeDtypeStruct((rows, NSAMPLE), jnp.int32),
        mesh=mesh,
        compiler_params=pltpu.CompilerParams(needs_layout_passes=False),
        scratch_types=[
            pltpu.VMEM((slab_rows, n), jnp.int32),
            pltpu.VMEM((rows_per_w, NSAMPLE), jnp.int32),
        ],
    )
    def sel(f_hbm, idx_hbm, slabv, bufv):
        wid = lax.axis_index("s") * SC_CORES + lax.axis_index("c")
        lanes = lax.broadcasted_iota(jnp.int32, (SC_LANES,), 0)
        zeros = jnp.zeros((SC_LANES,), jnp.int32)

        def slab_body(sl, _):
            pltpu.sync_copy(
                f_hbm.at[pl.ds(wid * rows_per_w + sl * slab_rows, slab_rows)],
                slabv)

            def row_body(j2, _):
                rowv = jnp.full((SC_LANES,), j2, jnp.int32)
                browv = sl * slab_rows + rowv
                unroll = 8
                nsup = nchunks // unroll

                def cond(carry):
                    sup, cntv = carry
                    return jnp.logical_and(sup < nsup,
                                           jnp.max(cntv) < NSAMPLE)

                def step(carry):
                    sup, cntv = carry
                    for k in range(unroll):
                        col = (sup * unroll + k) * SC_LANES + lanes
                        mi = plsc.load_gather(slabv, [rowv, col])
                        m = mi == 1
                        incl = plsc.cumsum(mi)
                        pos = cntv + incl - mi
                        mstore = jnp.logical_and(m, pos < NSAMPLE)
                        plsc.store_scatter(bufv, [browv, pos], col,
                                           mask=mstore)
                        cntv = cntv + plsc.all_reduce_population_count(m)
                    return (sup + 1, cntv)

                _, cntv = lax.while_loop(
                    cond, step,
                    (jnp.int32(0), jnp.zeros((SC_LANES,), jnp.int32)))
                cnt = jnp.max(cntv)

                firstv = plsc.load_gather(bufv, [browv, zeros])
                for mch in range(NSAMPLE // SC_LANES):
                    slots = mch * SC_LANES + lanes
                    cur = plsc.load_gather(bufv, [browv, slots])
                    plsc.store_scatter(bufv, [browv, slots],
                                       jnp.where(slots < cnt, cur, firstv))
                return 0

            lax.fori_loop(0, slab_rows, row_body, 0)
            return 0

        lax.fori_loop(0, nslabs, slab_body, 0)
        pltpu.sync_copy(bufv, idx_hbm.at[pl.ds(wid * rows_per_w, rows_per_w)])

    return sel(flags)


def _gather_sc(g1flat, idxflat):
    total, c1 = g1flat.shape[0], g1flat.shape[1]
    nrows = idxflat.shape[0]
    rows_per_w = nrows // SC_WORKERS
    chunk = 128
    inner = 8
    slab = chunk * inner
    outer = rows_per_w // slab
    mesh = plsc.VectorSubcoreMesh(core_axis_name="c", subcore_axis_name="s",
                                  num_cores=SC_CORES, num_subcores=SC_SUBCORES)

    @functools.partial(
        pl.kernel,
        out_type=jax.ShapeDtypeStruct((nrows, c1), jnp.float32),
        mesh=mesh,
        compiler_params=pltpu.CompilerParams(needs_layout_passes=False,
                                             use_tc_tiling_on_sc=False),
        scratch_types=[
            pltpu.VMEM((rows_per_w,), jnp.int32),
            pltpu.VMEM((slab, c1), jnp.float32),
            pltpu.SemaphoreType.DMA,
        ],
    )
    def gat(tab_hbm, idx_hbm, out_hbm, idxv, rowsv, sem):
        wid = lax.axis_index("s") * SC_CORES + lax.axis_index("c")
        base = wid * rows_per_w
        pltpu.sync_copy(idx_hbm.at[pl.ds(base, rows_per_w)], idxv)

        def outer_body(o, _):
            handles = []
            for k in range(inner):
                src = tab_hbm.at[idxv.at[pl.ds(o * slab + k * chunk, chunk)]]
                dst = rowsv.at[pl.ds(k * chunk, chunk)]
                handles.append(pltpu.async_copy(src, dst, sem))
            for h in handles:
                h.wait()
            pltpu.sync_copy(rowsv, out_hbm.at[pl.ds(base + o * slab, slab)])
            return 0

        lax.fori_loop(0, outer, outer_body, 0)

    return gat(g1flat, idxflat)


def _qrelu_triple(y):
    q2 = y[0] * y[0] + y[1] * y[1] + y[2] * y[2]
    coef = jnp.sqrt(jnp.minimum(q2, 1.0))
    return y * coef[None]


def _stats1_body(x_ref, u_ref, ss_ref):
    i = pl.program_id(0)
    j = pl.program_id(1)

    @pl.when(jnp.logical_and(i == 0, j == 0))
    def _():
        ss_ref[...] = jnp.zeros_like(ss_ref)

    x = x_ref[0]
    u = u_ref[0]
    sch, c = u.shape
    y = x.reshape(sch, NSAMPLE, c) + u[:, None, :]
    ss_ref[...] += jnp.sum(y * y, axis=(0, 1)).reshape(1, c)


def _stats1(x1, u1, sch):
    b3, nrows, c1 = x1.shape
    jgrid = NPOINT // sch
    return pl.pallas_call(
        _stats1_body,
        grid=(b3, jgrid),
        in_specs=[
            pl.BlockSpec((1, sch * NSAMPLE, c1), lambda i, j: (i, j, 0)),
            pl.BlockSpec((1, sch, c1), lambda i, j: (i, j, 0)),
        ],
        out_specs=pl.BlockSpec((1, c1), lambda i, j: (0, 0)),
        out_shape=jax.ShapeDtypeStruct((1, c1), jnp.float32),
    )(x1, u1)


def _layer2_body(x_ref, u_ref, ss1_ref, w2_ref, b2_ref, y2_ref, ss2_ref, *,
                 cnt1):
    b = pl.program_id(0)
    j = pl.program_id(1)

    @pl.when(jnp.logical_and(b == 0, j == 0))
    def _():
        ss2_ref[...] = jnp.zeros_like(ss2_ref)

    s1 = lax.rsqrt(ss1_ref[0] / cnt1 + 1e-5)
    x = x_ref[...]
    u = u_ref[...]
    _, sch, c1 = u.shape
    y1 = x.reshape(3, sch, NSAMPLE, c1) + u[:, :, None, :]
    y1 = y1 * s1[None, None, None, :]
    z1 = _qrelu_triple(y1).reshape(3, sch * NSAMPLE, c1)
    w2 = w2_ref[...]
    b2 = b2_ref[...]
    for c in range(3):
        y2 = lax.dot_general(z1[c].astype(jnp.bfloat16), w2,
                             (((1,), (1,)), ((), ())),
                             preferred_element_type=jnp.float32) + b2
        y2_ref[c] = y2.astype(jnp.bfloat16)
        ss2_ref[...] += jnp.sum(y2 * y2, axis=0).reshape(1, -1)


def _layer2(x1, u1, ss1, W2, b2, sch, cnt1):
    b3, nrows, c1 = x1.shape
    c2 = W2.shape[0]
    bsz = b3 // 3
    jgrid = NPOINT // sch
    return pl.pallas_call(
        functools.partial(_layer2_body, cnt1=cnt1),
        grid=(bsz, jgrid),
        in_specs=[
            pl.BlockSpec((3, sch * NSAMPLE, c1), lambda b, j: (b, j, 0)),
            pl.BlockSpec((3, sch, c1), lambda b, j: (b, j, 0)),
            pl.BlockSpec((1, c1), lambda b, j: (0, 0)),
            pl.BlockSpec((c2, c1), lambda b, j: (0, 0)),
            pl.BlockSpec((1, c2), lambda b, j: (0, 0)),
        ],
        out_specs=[
            pl.BlockSpec((3, sch * NSAMPLE, c2), lambda b, j: (b, j, 0)),
            pl.BlockSpec((1, c2), lambda b, j: (0, 0)),
        ],
        out_shape=[
            jax.ShapeDtypeStruct((b3, nrows, c2), jnp.bfloat16),
            jax.ShapeDtypeStruct((1, c2), jnp.float32),
        ],
    )(x1, u1, ss1, W2.astype(jnp.bfloat16), b2.reshape(1, c2))


def _stats3_body(y2_ref, ss2_ref, w3_ref, b3_ref, ss3_ref, *, cnt2):
    b = pl.program_id(0)
    j = pl.program_id(1)

    @pl.when(jnp.logical_and(b == 0, j == 0))
    def _():
        ss3_ref[...] = jnp.zeros_like(ss3_ref)

    s2 = lax.rsqrt(ss2_ref[0] / cnt2 + 1e-5)
    y2 = y2_ref[...].astype(jnp.float32)
    _, rows, c2 = y2.shape
    sch = rows // NSAMPLE
    y2 = y2.reshape(3, sch, NSAMPLE, c2) * s2[None, None, None, :]
    z2 = _qrelu_triple(y2).reshape(3, rows, c2)
    w3 = w3_ref[...]
    b3v = b3_ref[...]
    for c in range(3):
        y3 = lax.dot_general(z2[c].astype(jnp.bfloat16), w3,
                             (((1,), (1,)), ((), ())),
                             preferred_element_type=jnp.float32) + b3v
        ss3_ref[...] += jnp.sum(y3 * y3, axis=0).reshape(1, -1)


def _stats3(y2, ss2, W3, b3w, sch, cnt2):
    b3, nrows, c2 = y2.shape
    c3 = W3.shape[0]
    bsz = b3 // 3
    jgrid = NPOINT // sch
    return pl.pallas_call(
        functools.partial(_stats3_body, cnt2=cnt2),
        grid=(bsz, jgrid),
        in_specs=[
            pl.BlockSpec((3, sch * NSAMPLE, c2), lambda b, j: (b, j, 0)),
            pl.BlockSpec((1, c2), lambda b, j: (0, 0)),
            pl.BlockSpec((c3, c2), lambda b, j: (0, 0)),
            pl.BlockSpec((1, c3), lambda b, j: (0, 0)),
        ],
        out_specs=pl.BlockSpec((1, c3), lambda b, j: (0, 0)),
        out_shape=jax.ShapeDtypeStruct((1, c3), jnp.float32),
    )(y2, ss2, W3.astype(jnp.bfloat16), b3w.reshape(1, c3))


def _final_body(y2_ref, ss2_ref, ss3_ref, w3_ref, b3_ref, out_ref, *,
                cnt2, cnt3):
    s2 = lax.rsqrt(ss2_ref[0] / cnt2 + 1e-5)
    s3 = lax.rsqrt(ss3_ref[0] / cnt3 + 1e-5)
    y2 = y2_ref[...].astype(jnp.float32)
    _, rows, c2 = y2.shape
    sch = rows // NSAMPLE
    y2 = y2.reshape(3, sch, NSAMPLE, c2) * s2[None, None, None, :]
    z2 = _qrelu_triple(y2).reshape(3, rows, c2)
    w3 = w3_ref[...]
    b3v = b3_ref[...]
    c3 = w3.shape[0]
    y3l = []
    for c in range(3):
        y3 = lax.dot_general(z2[c].astype(jnp.bfloat16), w3,
                             (((1,), (1,)), ((), ())),
                             preferred_element_type=jnp.float32) + b3v
        y3l.append(y3.reshape(sch, NSAMPLE, c3))
    y3 = jnp.stack(y3l, axis=0) * s3[None, None, None, :]
    z3 = _qrelu_triple(y3)
    out_ref[...] = jnp.max(z3, axis=2)


def _final(y2, ss2, ss3, W3, b3w, sch, cnt2, cnt3):
    b3, nrows, c2 = y2.shape
    c3 = W3.shape[0]
    bsz = b3 // 3
    jgrid = NPOINT // sch
    return pl.pallas_call(
        functools.partial(_final_body, cnt2=cnt2, cnt3=cnt3),
        grid=(bsz, jgrid),
        in_specs=[
            pl.BlockSpec((3, sch * NSAMPLE, c2), lambda b, j: (b, j, 0)),
            pl.BlockSpec((1, c2), lambda b, j: (0, 0)),
            pl.BlockSpec((1, c3), lambda b, j: (0, 0)),
            pl.BlockSpec((c3, c2), lambda b, j: (0, 0)),
            pl.BlockSpec((1, c3), lambda b, j: (0, 0)),
        ],
        out_specs=pl.BlockSpec((3, sch, c3), lambda b, j: (b, j, 0)),
        out_shape=jax.ShapeDtypeStruct((b3, NPOINT, c3), jnp.float32),
    )(y2, ss2, ss3, W3.astype(jnp.bfloat16), b3w.reshape(1, c3))


def kernel(xyz, points, W1, b1, W2, b2, W3, b3):
    bsz, _, n = xyz.shape
    b3n, d, _ = points.shape
    c1 = W1.shape[0]

    cent, nxyz = _fps(xyz)

    g1, u1 = _g1_u1(xyz, points, nxyz, W1, b1)

    flags = _flags(jnp.transpose(nxyz, (0, 2, 1)),
                   jnp.transpose(xyz, (0, 2, 1)))
    idx = _select_sc(flags)

    offs = (jnp.arange(b3n, dtype=jnp.int32) * n).reshape(b3n, 1)
    idxb = jnp.broadcast_to(
        idx.reshape(bsz, 1, NPOINT * NSAMPLE), (bsz, 3, NPOINT * NSAMPLE)
    ).reshape(b3n, NPOINT * NSAMPLE) + offs
    x1 = _gather_sc(g1.reshape(b3n * n, c1), idxb.reshape(-1))
    x1 = x1.reshape(b3n, NPOINT * NSAMPLE, c1)

    sch = 64
    cnt = float(b3n * NPOINT * NSAMPLE)
    ss1 = _stats1(x1, u1, sch)
    y2, ss2 = _layer2(x1, u1, ss1, W2, b2, sch, cnt)
    ss3 = _stats3(y2, ss2, W3, b3, sch, cnt)
    new_points = _final(y2, ss2, ss3, W3, b3, sch, cnt, cnt)

    return nxyz, jnp.transpose(new_points, (0, 2, 1))


# speedup vs baseline: 13.7333x; 1.0100x over previous
import functools

import jax
import jax.numpy as jnp
import numpy as np
from jax import lax
from jax.experimental import pallas as pl
from jax.experimental.pallas import tpu as pltpu
from jax.experimental.pallas import tpu_sc as plsc

NPOINT = 512
RADIUS = 0.2
NSAMPLE = 64
R2 = float(RADIUS) ** 2

SC_CORES = 2
SC_SUBCORES = 16
SC_WORKERS = SC_CORES * SC_SUBCORES
SC_LANES = 16


def _fps_body(xyz_ref, cent_ref, nx_ref):
    x = xyz_ref[...]
    bsz, _, n = x.shape
    x0 = x[:, 0, :]
    x1 = x[:, 1, :]
    x2 = x[:, 2, :]
    m0 = jnp.mean(x0, axis=1, keepdims=True)
    m1 = jnp.mean(x1, axis=1, keepdims=True)
    m2 = jnp.mean(x2, axis=1, keepdims=True)
    dist0 = (x0 - m0) ** 2 + (x1 - m1) ** 2 + (x2 - m2) ** 2
    distance = jnp.minimum(jnp.full((bsz, n), 1e10, jnp.float32), dist0)
    lane = lax.broadcasted_iota(jnp.int32, (bsz, n), 1)
    scol = lax.broadcasted_iota(jnp.int32, (bsz, NPOINT), 1)

    def argmax_rows(d):
        mv = jnp.max(d, axis=1, keepdims=True)
        return jnp.min(jnp.where(d == mv, lane, n), axis=1, keepdims=True)

    far = argmax_rows(distance)
    cent = jnp.zeros((bsz, NPOINT), jnp.int32)
    nx0 = jnp.zeros((bsz, NPOINT), jnp.float32)
    nx1 = jnp.zeros((bsz, NPOINT), jnp.float32)
    nx2 = jnp.zeros((bsz, NPOINT), jnp.float32)

    lane3 = lax.broadcasted_iota(jnp.int32, (bsz, 3, n), 2)

    def body(i, carry):
        distance, far, cent, nx0, nx1, nx2 = carry
        oh3 = lane3 == far[:, None, :]
        c = jnp.sum(jnp.where(oh3, x, 0.0), axis=2, keepdims=True)
        c0 = c[:, 0]
        c1 = c[:, 1]
        c2 = c[:, 2]
        sel = scol == i
        cent = jnp.where(sel, far, cent)
        nx0 = jnp.where(sel, c0, nx0)
        nx1 = jnp.where(sel, c1, nx1)
        nx2 = jnp.where(sel, c2, nx2)
        d = (x0 - c0) ** 2 + (x1 - c1) ** 2 + (x2 - c2) ** 2
        distance = jnp.minimum(distance, d)
        far = argmax_rows(distance)
        return (distance, far, cent, nx0, nx1, nx2)

    carry = (distance, far, cent, nx0, nx1, nx2)
    _, _, cent, nx0, nx1, nx2 = lax.fori_loop(0, NPOINT, body, carry)
    cent_ref[...] = cent
    nx_ref[:, 0, :] = nx0
    nx_ref[:, 1, :] = nx1
    nx_ref[:, 2, :] = nx2


def _fps(xyz):
    bsz = xyz.shape[0]
    return pl.pallas_call(
        _fps_body,
        out_shape=[
            jax.ShapeDtypeStruct((bsz, NPOINT), jnp.int32),
            jax.ShapeDtypeStruct((bsz, 3, NPOINT), jnp.float32),
        ],
    )(xyz)


def _g1_body(xyz_ref, pts_ref, nx_ref, w1_ref, b1_ref, g1_ref, u1_ref):
    pts = pts_ref[0]
    xyzrow = xyz_ref[0, 0]
    nxrow = nx_ref[0, 0]
    w1 = w1_ref[...]
    b1 = b1_ref[...]
    w01 = w1[:, 0] + w1[:, 1]
    w1p = w1[:, 2:]
    g = lax.dot_general(pts, w1p, (((0,), (1,)), ((), ())),
                        preferred_element_type=jnp.float32)
    g = g + xyzrow[:, None] * w01[None, :]
    g1_ref[0] = g
    u1_ref[0] = b1 - nxrow[:, None] * w1[:, 0][None, :]


def _g1_u1(xyz, points, nxyz, W1, b1):
    b3, d, n = points.shape
    c1 = W1.shape[0]
    return pl.pallas_call(
        _g1_body,
        grid=(b3,),
        in_specs=[
            pl.BlockSpec((1, 1, n), lambda i: (i, 0, 0)),
            pl.BlockSpec((1, d, n), lambda i: (i, 0, 0)),
            pl.BlockSpec((1, 1, NPOINT), lambda i: (i, 0, 0)),
            pl.BlockSpec((c1, d + 2), lambda i: (0, 0)),
            pl.BlockSpec((1, c1), lambda i: (0, 0)),
        ],
        out_specs=[
            pl.BlockSpec((1, n, c1), lambda i: (i, 0, 0)),
            pl.BlockSpec((1, NPOINT, c1), lambda i: (i, 0, 0)),
        ],
        out_shape=[
            jax.ShapeDtypeStruct((b3, n, c1), jnp.float32),
            jax.ShapeDtypeStruct((b3, NPOINT, c1), jnp.float32),
        ],
    )(xyz.reshape(b3, 1, n), points, nxyz.reshape(b3, 1, NPOINT),
      W1, b1.reshape(1, c1))


def _flags_body(nx_ref, xt_ref, f_ref):
    src = nx_ref[0]
    dst = xt_ref[0]
    mm = lax.dot_general(src, dst, (((1,), (1,)), ((), ())))
    dist = -2.0 * mm
    dist = dist + jnp.sum(src ** 2, -1)[:, None]
    dist = dist + jnp.sum(dst ** 2, -1)[None, :]
    f_ref[0] = (dist <= R2).astype(jnp.int32)


def _flags(nx_t, xyz_t):
    bsz, n, _ = xyz_t.shape
    return pl.pallas_call(
        _flags_body,
        grid=(bsz,),
        in_specs=[pl.BlockSpec((1, NPOINT, 3), lambda i: (i, 0, 0)),
                  pl.BlockSpec((1, n, 3), lambda i: (i, 0, 0))],
        out_specs=pl.BlockSpec((1, NPOINT, n), lambda i: (i, 0, 0)),
        out_shape=jax.ShapeDtypeStruct((bsz, NPOINT, n), jnp.int32),
    )(nx_t, xyz_t)


def _select_sc(flags):
    rows = flags.shape[0] * flags.shape[1]
    n = flags.shape[2]
    flags = flags.reshape(rows, n)
    rows_per_w = rows // SC_WORKERS
    slab_rows = 16
    nslabs = rows_per_w // slab_rows
    nchunks = n // SC_LANES
    mesh = plsc.VectorSubcoreMesh(core_axis_name="c", subcore_axis_name="s",
                                  num_cores=SC_CORES, num_subcores=SC_SUBCORES)

    @functools.partial(
        pl.kernel,
        out_type=jax.ShapeDtypeStruct((rows, NSAMPLE), jnp.int32),
        mesh=mesh,
        compiler_params=pltpu.CompilerParams(needs_layout_passes=False),
        scratch_types=[
            pltpu.VMEM((slab_rows, n), jnp.int32),
            pltpu.VMEM((rows_per_w, NSAMPLE), jnp.int32),
        ],
    )
    def sel(f_hbm, idx_hbm, slabv, bufv):
        wid = lax.axis_index("s") * SC_CORES + lax.axis_index("c")
        lanes = lax.broadcasted_iota(jnp.int32, (SC_LANES,), 0)
        zeros = jnp.zeros((SC_LANES,), jnp.int32)

        def slab_body(sl, _):
            pltpu.sync_copy(
                f_hbm.at[pl.ds(wid * rows_per_w + sl * slab_rows, slab_rows)],
                slabv)

            def row_body(j2, _):
                rowv = jnp.full((SC_LANES,), j2, jnp.int32)
                browv = sl * slab_rows + rowv
                unroll = 8
                nsup = nchunks // unroll

                def cond(carry):
                    sup, cntv = carry
                    return jnp.logical_and(sup < nsup,
                                           jnp.max(cntv) < NSAMPLE)

                def step(carry):
                    sup, cntv = carry
                    for k in range(unroll):
                        col = (sup * unroll + k) * SC_LANES + lanes
                        mi = plsc.load_gather(slabv, [rowv, col])
                        m = mi == 1
                        incl = plsc.cumsum(mi)
                        pos = cntv + incl - mi
                        mstore = jnp.logical_and(m, pos < NSAMPLE)
                        plsc.store_scatter(bufv, [browv, pos], col,
                                           mask=mstore)
                        cntv = cntv + plsc.all_reduce_population_count(m)
                    return (sup + 1, cntv)

                _, cntv = lax.while_loop(
                    cond, step,
                    (jnp.int32(0), jnp.zeros((SC_LANES,), jnp.int32)))
                cnt = jnp.max(cntv)

                firstv = plsc.load_gather(bufv, [browv, zeros])
                for mch in range(NSAMPLE // SC_LANES):
                    slots = mch * SC_LANES + lanes
                    cur = plsc.load_gather(bufv, [browv, slots])
                    plsc.store_scatter(bufv, [browv, slots],
                                       jnp.where(slots < cnt, cur, firstv))
                return 0

            lax.fori_loop(0, slab_rows, row_body, 0)
            return 0

        lax.fori_loop(0, nslabs, slab_body, 0)
        pltpu.sync_copy(bufv, idx_hbm.at[pl.ds(wid * rows_per_w, rows_per_w)])

    return sel(flags)


def _gather_sc(g1flat, idxflat):
    total, c1 = g1flat.shape[0], g1flat.shape[1]
    nrows = idxflat.shape[0]
    rows_per_w = nrows // SC_WORKERS
    chunk = 128
    inner = 8
    slab = chunk * inner
    outer = rows_per_w // slab
    mesh = plsc.VectorSubcoreMesh(core_axis_name="c", subcore_axis_name="s",
                                  num_cores=SC_CORES, num_subcores=SC_SUBCORES)

    @functools.partial(
        pl.kernel,
        out_type=jax.ShapeDtypeStruct((nrows, c1), jnp.float32),
        mesh=mesh,
        compiler_params=pltpu.CompilerParams(needs_layout_passes=False,
                                             use_tc_tiling_on_sc=False),
        scratch_types=[
            pltpu.VMEM((rows_per_w,), jnp.int32),
            pltpu.VMEM((slab, c1), jnp.float32),
            pltpu.SemaphoreType.DMA,
        ],
    )
    def gat(tab_hbm, idx_hbm, out_hbm, idxv, rowsv, sem):
        wid = lax.axis_index("s") * SC_CORES + lax.axis_index("c")
        base = wid * rows_per_w
        pltpu.sync_copy(idx_hbm.at[pl.ds(base, rows_per_w)], idxv)

        def outer_body(o, _):
            handles = []
            for k in range(inner):
                src = tab_hbm.at[idxv.at[pl.ds(o * slab + k * chunk, chunk)]]
                dst = rowsv.at[pl.ds(k * chunk, chunk)]
                handles.append(pltpu.async_copy(src, dst, sem))
            for h in handles:
                h.wait()
            pltpu.sync_copy(rowsv, out_hbm.at[pl.ds(base + o * slab, slab)])
            return 0

        lax.fori_loop(0, outer, outer_body, 0)

    return gat(g1flat, idxflat)


def _qrelu_triple(y):
    q2 = y[0] * y[0] + y[1] * y[1] + y[2] * y[2]
    coef = jnp.sqrt(jnp.minimum(q2, 1.0))
    return y * coef[None]


def _stats1_body(x_ref, u_ref, ss_ref):
    i = pl.program_id(0)
    j = pl.program_id(1)

    @pl.when(jnp.logical_and(i == 0, j == 0))
    def _():
        ss_ref[...] = jnp.zeros_like(ss_ref)

    x = x_ref[0]
    u = u_ref[0]
    sch, c = u.shape
    y = x.reshape(sch, NSAMPLE, c) + u[:, None, :]
    ss_ref[...] += jnp.sum(y * y, axis=(0, 1)).reshape(1, c)


def _stats1(x1, u1, sch):
    b3, nrows, c1 = x1.shape
    jgrid = NPOINT // sch
    return pl.pallas_call(
        _stats1_body,
        grid=(b3, jgrid),
        in_specs=[
            pl.BlockSpec((1, sch * NSAMPLE, c1), lambda i, j: (i, j, 0)),
            pl.BlockSpec((1, sch, c1), lambda i, j: (i, j, 0)),
        ],
        out_specs=pl.BlockSpec((1, c1), lambda i, j: (0, 0)),
        out_shape=jax.ShapeDtypeStruct((1, c1), jnp.float32),
    )(x1, u1)


def _layer2_body(x_ref, u_ref, ss1_ref, w2_ref, b2_ref, y2_ref, ss2_ref, *,
                 cnt1):
    b = pl.program_id(0)
    j = pl.program_id(1)

    @pl.when(jnp.logical_and(b == 0, j == 0))
    def _():
        ss2_ref[...] = jnp.zeros_like(ss2_ref)

    s1 = lax.rsqrt(ss1_ref[0] / cnt1 + 1e-5)
    x = x_ref[...]
    u = u_ref[...]
    _, sch, c1 = u.shape
    y1 = x.reshape(3, sch, NSAMPLE, c1) + u[:, :, None, :]
    y1 = y1 * s1[None, None, None, :]
    z1 = _qrelu_triple(y1).reshape(3, sch * NSAMPLE, c1)
    w2 = w2_ref[...]
    b2 = b2_ref[...]
    for c in range(3):
        y2 = lax.dot_general(z1[c].astype(jnp.bfloat16), w2,
                             (((1,), (1,)), ((), ())),
                             preferred_element_type=jnp.float32) + b2
        y2_ref[c] = y2.astype(jnp.bfloat16)
        ss2_ref[...] += jnp.sum(y2 * y2, axis=0).reshape(1, -1)


def _layer2(x1, u1, ss1, W2, b2, sch, cnt1):
    b3, nrows, c1 = x1.shape
    c2 = W2.shape[0]
    bsz = b3 // 3
    jgrid = NPOINT // sch
    return pl.pallas_call(
        functools.partial(_layer2_body, cnt1=cnt1),
        grid=(bsz, jgrid),
        in_specs=[
            pl.BlockSpec((3, sch * NSAMPLE, c1), lambda b, j: (b, j, 0)),
            pl.BlockSpec((3, sch, c1), lambda b, j: (b, j, 0)),
            pl.BlockSpec((1, c1), lambda b, j: (0, 0)),
            pl.BlockSpec((c2, c1), lambda b, j: (0, 0)),
            pl.BlockSpec((1, c2), lambda b, j: (0, 0)),
        ],
        out_specs=[
            pl.BlockSpec((3, sch * NSAMPLE, c2), lambda b, j: (b, j, 0)),
            pl.BlockSpec((1, c2), lambda b, j: (0, 0)),
        ],
        out_shape=[
            jax.ShapeDtypeStruct((b3, nrows, c2), jnp.bfloat16),
            jax.ShapeDtypeStruct((1, c2), jnp.float32),
        ],
    )(x1, u1, ss1, W2.astype(jnp.bfloat16), b2.reshape(1, c2))


def _stats3_body(y2_ref, ss2_ref, w3_ref, b3_ref, ss3_ref, *, cnt2):
    b = pl.program_id(0)
    j = pl.program_id(1)

    @pl.when(jnp.logical_and(b == 0, j == 0))
    def _():
        ss3_ref[...] = jnp.zeros_like(ss3_ref)

    s2 = lax.rsqrt(ss2_ref[0] / cnt2 + 1e-5)
    y2 = y2_ref[...].astype(jnp.float32)
    _, rows, c2 = y2.shape
    sch = rows // NSAMPLE
    y2 = y2.reshape(3, sch, NSAMPLE, c2) * s2[None, None, None, :]
    z2 = _qrelu_triple(y2).reshape(3, rows, c2)
    w3 = w3_ref[...]
    b3v = b3_ref[...]
    for c in range(3):
        y3 = lax.dot_general(z2[c].astype(jnp.bfloat16), w3,
                             (((1,), (1,)), ((), ())),
                             preferred_element_type=jnp.float32) + b3v
        ss3_ref[...] += jnp.sum(y3 * y3, axis=0).reshape(1, -1)


def _stats3(y2, ss2, W3, b3w, sch, cnt2):
    b3, nrows, c2 = y2.shape
    c3 = W3.shape[0]
    bsz = b3 // 3
    jgrid = NPOINT // sch
    return pl.pallas_call(
        functools.partial(_stats3_body, cnt2=cnt2),
        grid=(bsz, jgrid),
        in_specs=[
            pl.BlockSpec((3, sch * NSAMPLE, c2), lambda b, j: (b, j, 0)),
            pl.BlockSpec((1, c2), lambda b, j: (0, 0)),
            pl.BlockSpec((c3, c2), lambda b, j: (0, 0)),
            pl.BlockSpec((1, c3), lambda b, j: (0, 0)),
        ],
        out_specs=pl.BlockSpec((1, c3), lambda b, j: (0, 0)),
        out_shape=jax.ShapeDtypeStruct((1, c3), jnp.float32),
    )(y2, ss2, W3.astype(jnp.bfloat16), b3w.reshape(1, c3))


def _final_body(y2_ref, ss2_ref, ss3_ref, w3_ref, b3_ref, out_ref, *,
                cnt2, cnt3):
    s2 = lax.rsqrt(ss2_ref[0] / cnt2 + 1e-5)
    s3 = lax.rsqrt(ss3_ref[0] / cnt3 + 1e-5)
    y2 = y2_ref[...].astype(jnp.float32)
    _, rows, c2 = y2.shape
    sch = rows // NSAMPLE
    y2 = y2.reshape(3, sch, NSAMPLE, c2) * s2[None, None, None, :]
    z2 = _qrelu_triple(y2).reshape(3, rows, c2)
    w3 = w3_ref[...]
    b3v = b3_ref[...]
    c3 = w3.shape[0]
    y3l = []
    for c in range(3):
        y3 = lax.dot_general(z2[c].astype(jnp.bfloat16), w3,
                             (((1,), (1,)), ((), ())),
                             preferred_element_type=jnp.float32) + b3v
        y3l.append(y3.reshape(sch, NSAMPLE, c3))
    y3 = jnp.stack(y3l, axis=0) * s3[None, None, None, :]
    z3 = _qrelu_triple(y3)
    out_ref[...] = jnp.max(z3, axis=2)


def _final(y2, ss2, ss3, W3, b3w, sch, cnt2, cnt3):
    b3, nrows, c2 = y2.shape
    c3 = W3.shape[0]
    bsz = b3 // 3
    jgrid = NPOINT // sch
    return pl.pallas_call(
        functools.partial(_final_body, cnt2=cnt2, cnt3=cnt3),
        grid=(bsz, jgrid),
        in_specs=[
            pl.BlockSpec((3, sch * NSAMPLE, c2), lambda b, j: (b, j, 0)),
            pl.BlockSpec((1, c2), lambda b, j: (0, 0)),
            pl.BlockSpec((1, c3), lambda b, j: (0, 0)),
            pl.BlockSpec((c3, c2), lambda b, j: (0, 0)),
            pl.BlockSpec((1, c3), lambda b, j: (0, 0)),
        ],
        out_specs=pl.BlockSpec((3, sch, c3), lambda b, j: (b, j, 0)),
        out_shape=jax.ShapeDtypeStruct((b3, NPOINT, c3), jnp.float32),
    )(y2, ss2, ss3, W3.astype(jnp.bfloat16), b3w.reshape(1, c3))


def kernel(xyz, points, W1, b1, W2, b2, W3, b3):
    bsz, _, n = xyz.shape
    b3n, d, _ = points.shape
    c1 = W1.shape[0]

    cent, nxyz = _fps(xyz)

    g1, u1 = _g1_u1(xyz, points, nxyz, W1, b1)

    flags = _flags(jnp.transpose(nxyz, (0, 2, 1)),
                   jnp.transpose(xyz, (0, 2, 1)))
    idx = _select_sc(flags)

    offs = (jnp.arange(b3n, dtype=jnp.int32) * n).reshape(b3n, 1)
    idxb = jnp.broadcast_to(
        idx.reshape(bsz, 1, NPOINT * NSAMPLE), (bsz, 3, NPOINT * NSAMPLE)
    ).reshape(b3n, NPOINT * NSAMPLE) + offs
    x1 = _gather_sc(g1.reshape(b3n * n, c1), idxb.reshape(-1))
    x1 = x1.reshape(b3n, NPOINT * NSAMPLE, c1)

    sch = 64
    cnt = float(b3n * NPOINT * NSAMPLE)
    ss1 = _stats1(x1, u1, sch)
    y2, ss2 = _layer2(x1, u1, ss1, W2, b2, sch, cnt)
    ss3 = _stats3(y2, ss2, W3, b3, sch, cnt)
    new_points = _final(y2, ss2, ss3, W3, b3, sch, cnt, cnt)

    return nxyz, jnp.transpose(new_points, (0, 2, 1))
